# Initial kernel scaffold; baseline (speedup 1.0000x reference)
import jax
import jax.numpy as jnp
from jax.experimental import pallas as pl


def kernel(x, edge_index, W1, b1, W2, b2):
    raise NotImplementedError("write your pallas kernel here")


# speedup vs baseline: 32.0368x; 32.0368x over previous
import functools

import jax
import jax.numpy as jnp
from jax import lax
from jax.experimental import pallas as pl
from jax.experimental.pallas import tpu as pltpu
from jax.experimental.pallas import tpu_sc as plsc

D = 16
CHUNK = 128
NCORES = 2
NSUB = 16
NTILES = NCORES * NSUB

_mesh = plsc.VectorSubcoreMesh(core_axis_name="c", subcore_axis_name="s")
_sc_params = pltpu.CompilerParams(use_tc_tiling_on_sc=False)


def _pad_rows(n):
    return ((n + 1 + NSUB * 8 - 1) // (NSUB * 8)) * (NSUB * 8)


@functools.partial(jax.jit, static_argnames=("n", "k"))
def _deg_pass(dstp, *, n, k):
    np_ = _pad_rows(n)
    rps = np_ // NSUB

    @functools.partial(
        pl.kernel,
        mesh=_mesh,
        compiler_params=_sc_params,
        out_type=jax.ShapeDtypeStruct((NCORES, np_, D), jnp.float32),
        scratch_types=[
            pltpu.VMEM((k, CHUNK), jnp.int32),
            pltpu.VMEM((CHUNK, D), jnp.float32),
            pltpu.VMEM((rps, D), jnp.float32),
            pltpu.VMEM_SHARED((np_, D), jnp.float32),
        ],
    )
    def deg_kernel(dst_hbm, out_hbm, dst_v, ones_v, zbuf_v, acc):
        cid = lax.axis_index("c")
        sid = lax.axis_index("s")
        wid = cid * NSUB + sid

        @pl.loop(0, rps)
        def _(r):
            zbuf_v[r, :] = jnp.zeros((D,), jnp.float32)

        @pl.loop(0, CHUNK)
        def _(r):
            ones_v[r, :] = jnp.ones((D,), jnp.float32)

        pltpu.sync_copy(zbuf_v, acc.at[pl.ds(sid * rps, rps)])
        plsc.subcore_barrier()

        pltpu.sync_copy(dst_hbm.at[wid], dst_v)

        @pl.loop(0, k)
        def _(j):
            pltpu.sync_copy(ones_v, acc.at[dst_v.at[j]], add=True)

        plsc.subcore_barrier()
        pltpu.sync_copy(
            acc.at[pl.ds(sid * rps, rps)],
            out_hbm.at[cid, pl.ds(sid * rps, rps)],
        )

    return deg_kernel(dstp)


@functools.partial(jax.jit, static_argnames=("n", "k"))
def _gather_scatter_pass(y, srcp, dstp, *, n, k):
    np_ = _pad_rows(n)
    rps = np_ // NSUB

    @functools.partial(
        pl.kernel,
        mesh=_mesh,
        compiler_params=_sc_params,
        out_type=jax.ShapeDtypeStruct((NCORES, np_, D), jnp.float32),
        scratch_types=[
            pltpu.VMEM((k, CHUNK), jnp.int32),
            pltpu.VMEM((k, CHUNK), jnp.int32),
            pltpu.VMEM((CHUNK, D), jnp.float32),
            pltpu.VMEM((CHUNK, D), jnp.float32),
            pltpu.VMEM((rps, D), jnp.float32),
            pltpu.VMEM_SHARED((np_, D), jnp.float32),
            pltpu.SemaphoreType.DMA,
            pltpu.SemaphoreType.DMA,
        ],
    )
    def scat_kernel(y_hbm, src_hbm, dst_hbm, out_hbm, src_v, dst_v,
                    rows0_v, rows1_v, zbuf_v, acc, sem0, sem1):
        cid = lax.axis_index("c")
        sid = lax.axis_index("s")
        wid = cid * NSUB + sid

        @pl.loop(0, rps)
        def _(r):
            zbuf_v[r, :] = jnp.zeros((D,), jnp.float32)

        pltpu.sync_copy(zbuf_v, acc.at[pl.ds(sid * rps, rps)])
        plsc.subcore_barrier()

        pltpu.sync_copy(src_hbm.at[wid], src_v)
        pltpu.sync_copy(dst_hbm.at[wid], dst_v)

        pltpu.async_copy(y_hbm.at[src_v.at[0]], rows0_v, sem0).wait()

        @pl.loop(0, (k - 1) // 2)
        def _(jj):
            j = jj * 2
            pltpu.async_copy(y_hbm.at[src_v.at[j + 1]], rows1_v, sem1)
            pltpu.sync_copy(rows0_v, acc.at[dst_v.at[j]], add=True)
            pltpu.make_async_copy(y_hbm.at[src_v.at[j + 1]], rows1_v, sem1).wait()
            pltpu.async_copy(y_hbm.at[src_v.at[j + 2]], rows0_v, sem0)
            pltpu.sync_copy(rows1_v, acc.at[dst_v.at[j + 1]], add=True)
            pltpu.make_async_copy(y_hbm.at[src_v.at[j + 2]], rows0_v, sem0).wait()

        pltpu.sync_copy(rows0_v, acc.at[dst_v.at[k - 1]], add=True)

        plsc.subcore_barrier()
        pltpu.sync_copy(
            acc.at[pl.ds(sid * rps, rps)],
            out_hbm.at[cid, pl.ds(sid * rps, rps)],
        )

    return scat_kernel(y, srcp, dstp)


def _mm_body(x_ref, w_ref, o_ref):
    o_ref[...] = jnp.dot(x_ref[...], w_ref[...],
                         preferred_element_type=jnp.float32)


def _deg_fin_body(d0_ref, d1_ref, xw_ref, y_ref, dis_ref, dinv_ref):
    deg = d0_ref[...] + d1_ref[...] + 1.0
    dis = lax.rsqrt(deg)
    dis_ref[...] = dis
    dinv_ref[...] = 1.0 / deg
    y_ref[...] = xw_ref[...] * dis


def _layer1_body(s0_ref, s1_ref, xw_ref, dis_ref, dinv_ref, b_ref, w2_ref,
                 y2_ref, hw2_ref):
    dis = dis_ref[...]
    pre = dis * (s0_ref[...] + s1_ref[...]) + xw_ref[...] * dinv_ref[...] \
        + b_ref[...]
    h = jnp.maximum(pre, 0.0)
    hw2 = jnp.dot(h, w2_ref[...], preferred_element_type=jnp.float32)
    hw2_ref[...] = hw2
    y2_ref[...] = hw2 * dis


def _layer2_body(s0_ref, s1_ref, hw2_ref, dis_ref, dinv_ref, b_ref, o_ref):
    o_ref[...] = dis_ref[...] * (s0_ref[...] + s1_ref[...]) \
        + hw2_ref[...] * dinv_ref[...] + b_ref[...]


def kernel(x, edge_index, W1, b1, W2, b2):
    n = x.shape[0]
    e = edge_index.shape[1]
    np_ = _pad_rows(n)
    k = -(-e // (NTILES * CHUNK))
    if k % 2 == 0:
        k += 1
    per_tile = k * CHUNK
    ep = NTILES * per_tile

    src = edge_index[0]
    dst = edge_index[1]
    pad = ep - e
    srcp = jnp.concatenate([src, jnp.zeros((pad,), jnp.int32)])
    dstp = jnp.concatenate([dst, jnp.full((pad,), n, jnp.int32)])
    srcp = srcp.reshape(NTILES, k, CHUNK)
    dstp = dstp.reshape(NTILES, k, CHUNK)

    fshape = jax.ShapeDtypeStruct((n, D), jnp.float32)

    degp = _deg_pass(dstp, n=n, k=k)
    xw1 = pl.pallas_call(_mm_body, out_shape=fshape)(x, W1)

    y1, dis, dinv = pl.pallas_call(
        _deg_fin_body, out_shape=(fshape, fshape, fshape)
    )(degp[0, :n], degp[1, :n], xw1)

    s1p = _gather_scatter_pass(y1, srcp, dstp, n=n, k=k)

    y2, hw2 = pl.pallas_call(
        _layer1_body, out_shape=(fshape, fshape)
    )(s1p[0, :n], s1p[1, :n], xw1, dis, dinv, b1.reshape(1, D), W2)

    s2p = _gather_scatter_pass(y2, srcp, dstp, n=n, k=k)

    out = pl.pallas_call(
        _layer2_body, out_shape=fshape
    )(s2p[0, :n], s2p[1, :n], hw2, dis, dinv, b2.reshape(1, D))
    return out


# speedup vs baseline: 48.2247x; 1.5053x over previous
import functools

import jax
import jax.numpy as jnp
from jax import lax
from jax.experimental import pallas as pl
from jax.experimental.pallas import tpu as pltpu
from jax.experimental.pallas import tpu_sc as plsc

D = 16
CHUNK = 128
NCORES = 2
NSUB = 16
NTILES = NCORES * NSUB

_mesh = plsc.VectorSubcoreMesh(core_axis_name="c", subcore_axis_name="s")
_sc_params = pltpu.CompilerParams(use_tc_tiling_on_sc=False)


def _pad_rows(n):
    return ((n + 1 + NSUB * 8 - 1) // (NSUB * 8)) * (NSUB * 8)


@functools.partial(jax.jit, static_argnames=("n", "k"))
def _deg_pass(dstp, *, n, k):
    np_ = _pad_rows(n)
    rps = np_ // NSUB

    @functools.partial(
        pl.kernel,
        mesh=_mesh,
        compiler_params=_sc_params,
        out_type=jax.ShapeDtypeStruct((NCORES, np_, D), jnp.float32),
        scratch_types=[
            pltpu.VMEM((k, CHUNK), jnp.int32),
            pltpu.VMEM((CHUNK, D), jnp.float32),
            pltpu.VMEM((rps, D), jnp.float32),
            pltpu.VMEM_SHARED((np_, D), jnp.float32),
        ],
    )
    def deg_kernel(dst_hbm, out_hbm, dst_v, ones_v, zbuf_v, acc):
        cid = lax.axis_index("c")
        sid = lax.axis_index("s")
        wid = cid * NSUB + sid

        @pl.loop(0, rps)
        def _(r):
            zbuf_v[r, :] = jnp.zeros((D,), jnp.float32)

        @pl.loop(0, CHUNK)
        def _(r):
            ones_v[r, :] = jnp.ones((D,), jnp.float32)

        pltpu.sync_copy(zbuf_v, acc.at[pl.ds(sid * rps, rps)])
        plsc.subcore_barrier()

        pltpu.sync_copy(dst_hbm.at[wid], dst_v)

        @pl.loop(0, k)
        def _(j):
            pltpu.sync_copy(ones_v, acc.at[dst_v.at[j]], add=True)

        plsc.subcore_barrier()
        pltpu.sync_copy(
            acc.at[pl.ds(sid * rps, rps)],
            out_hbm.at[cid, pl.ds(sid * rps, rps)],
        )

    return deg_kernel(dstp)


@functools.partial(jax.jit, static_argnames=("n", "k"))
def _gather_scatter_pass(y, srcp, dstp, *, n, k):
    np_ = _pad_rows(n)
    rps = np_ // NSUB

    stage = n // NSUB
    tail = n - NSUB * stage

    @functools.partial(
        pl.kernel,
        mesh=_mesh,
        compiler_params=_sc_params,
        out_type=jax.ShapeDtypeStruct((NCORES, np_, D), jnp.float32),
        scratch_types=[
            pltpu.VMEM((k, CHUNK), jnp.int32),
            pltpu.VMEM((k, CHUNK), jnp.int32),
            pltpu.VMEM((CHUNK, D), jnp.float32),
            pltpu.VMEM((CHUNK, D), jnp.float32),
            pltpu.VMEM((rps, D), jnp.float32),
            pltpu.VMEM_SHARED((n, D), jnp.float32),
            pltpu.VMEM_SHARED((np_, D), jnp.float32),
            pltpu.SemaphoreType.DMA,
            pltpu.SemaphoreType.DMA,
        ],
    )
    def scat_kernel(y_hbm, src_hbm, dst_hbm, out_hbm, src_v, dst_v,
                    rows0_v, rows1_v, zbuf_v, y_spm, acc, sem0, sem1):
        cid = lax.axis_index("c")
        sid = lax.axis_index("s")
        wid = cid * NSUB + sid

        pltpu.sync_copy(y_hbm.at[pl.ds(sid * stage, stage)],
                        y_spm.at[pl.ds(sid * stage, stage)])
        if tail:
            @pl.when(sid == 0)
            def _():
                pltpu.sync_copy(y_hbm.at[pl.ds(NSUB * stage, tail)],
                                y_spm.at[pl.ds(NSUB * stage, tail)])

        @pl.loop(0, rps)
        def _(r):
            zbuf_v[r, :] = jnp.zeros((D,), jnp.float32)

        pltpu.sync_copy(zbuf_v, acc.at[pl.ds(sid * rps, rps)])
        plsc.subcore_barrier()

        pltpu.sync_copy(src_hbm.at[wid], src_v)
        pltpu.sync_copy(dst_hbm.at[wid], dst_v)

        pltpu.async_copy(y_spm.at[src_v.at[0]], rows0_v, sem0).wait()

        @pl.loop(0, (k - 1) // 2)
        def _(jj):
            j = jj * 2
            pltpu.async_copy(y_spm.at[src_v.at[j + 1]], rows1_v, sem1)
            pltpu.sync_copy(rows0_v, acc.at[dst_v.at[j]], add=True)
            pltpu.make_async_copy(y_spm.at[src_v.at[j + 1]], rows1_v, sem1).wait()
            pltpu.async_copy(y_spm.at[src_v.at[j + 2]], rows0_v, sem0)
            pltpu.sync_copy(rows1_v, acc.at[dst_v.at[j + 1]], add=True)
            pltpu.make_async_copy(y_spm.at[src_v.at[j + 2]], rows0_v, sem0).wait()

        pltpu.sync_copy(rows0_v, acc.at[dst_v.at[k - 1]], add=True)

        plsc.subcore_barrier()
        pltpu.sync_copy(
            acc.at[pl.ds(sid * rps, rps)],
            out_hbm.at[cid, pl.ds(sid * rps, rps)],
        )

    return scat_kernel(y, srcp, dstp)


def _mm_body(x_ref, w_ref, o_ref):
    o_ref[...] = jnp.dot(x_ref[...], w_ref[...],
                         preferred_element_type=jnp.float32)


def _deg_fin_body(d0_ref, d1_ref, xw_ref, y_ref, dis_ref, dinv_ref):
    deg = d0_ref[...] + d1_ref[...] + 1.0
    dis = lax.rsqrt(deg)
    dis_ref[...] = dis
    dinv_ref[...] = 1.0 / deg
    y_ref[...] = xw_ref[...] * dis


def _layer1_body(s0_ref, s1_ref, xw_ref, dis_ref, dinv_ref, b_ref, w2_ref,
                 y2_ref, hw2_ref):
    dis = dis_ref[...]
    pre = dis * (s0_ref[...] + s1_ref[...]) + xw_ref[...] * dinv_ref[...] \
        + b_ref[...]
    h = jnp.maximum(pre, 0.0)
    hw2 = jnp.dot(h, w2_ref[...], preferred_element_type=jnp.float32)
    hw2_ref[...] = hw2
    y2_ref[...] = hw2 * dis


def _layer2_body(s0_ref, s1_ref, hw2_ref, dis_ref, dinv_ref, b_ref, o_ref):
    o_ref[...] = dis_ref[...] * (s0_ref[...] + s1_ref[...]) \
        + hw2_ref[...] * dinv_ref[...] + b_ref[...]


def kernel(x, edge_index, W1, b1, W2, b2):
    n = x.shape[0]
    e = edge_index.shape[1]
    np_ = _pad_rows(n)
    k = -(-e // (NTILES * CHUNK))
    if k % 2 == 0:
        k += 1
    per_tile = k * CHUNK
    ep = NTILES * per_tile

    src = edge_index[0]
    dst = edge_index[1]
    pad = ep - e
    srcp = jnp.concatenate([src, jnp.zeros((pad,), jnp.int32)])
    dstp = jnp.concatenate([dst, jnp.full((pad,), n, jnp.int32)])
    srcp = srcp.reshape(NTILES, k, CHUNK)
    dstp = dstp.reshape(NTILES, k, CHUNK)

    fshape = jax.ShapeDtypeStruct((n, D), jnp.float32)

    degp = _deg_pass(dstp, n=n, k=k)
    xw1 = pl.pallas_call(_mm_body, out_shape=fshape)(x, W1)

    y1, dis, dinv = pl.pallas_call(
        _deg_fin_body, out_shape=(fshape, fshape, fshape)
    )(degp[0, :n], degp[1, :n], xw1)

    s1p = _gather_scatter_pass(y1, srcp, dstp, n=n, k=k)

    y2, hw2 = pl.pallas_call(
        _layer1_body, out_shape=(fshape, fshape)
    )(s1p[0, :n], s1p[1, :n], xw1, dis, dinv, b1.reshape(1, D), W2)

    s2p = _gather_scatter_pass(y2, srcp, dstp, n=n, k=k)

    out = pl.pallas_call(
        _layer2_body, out_shape=fshape
    )(s2p[0, :n], s2p[1, :n], hw2, dis, dinv, b2.reshape(1, D))
    return out


# speedup vs baseline: 49.6276x; 1.0291x over previous
import functools

import jax
import jax.numpy as jnp
from jax import lax
from jax.experimental import pallas as pl
from jax.experimental.pallas import tpu as pltpu
from jax.experimental.pallas import tpu_sc as plsc

D = 16
CHUNK = 128
NCORES = 2
NSUB = 16
NTILES = NCORES * NSUB

_mesh = plsc.VectorSubcoreMesh(core_axis_name="c", subcore_axis_name="s")
_sc_params = pltpu.CompilerParams(use_tc_tiling_on_sc=False)
_sc_params_nl = pltpu.CompilerParams(use_tc_tiling_on_sc=False,
                                     needs_layout_passes=False)


def _pad_rows(n):
    return ((n + 1 + NSUB * 8 - 1) // (NSUB * 8)) * (NSUB * 8)


@functools.partial(jax.jit, static_argnames=("n", "k"))
def _deg_pass(dstp, *, n, k):
    np_ = _pad_rows(n)
    rps = np_ // NSUB

    @functools.partial(
        pl.kernel,
        mesh=_mesh,
        compiler_params=_sc_params,
        out_type=jax.ShapeDtypeStruct((NCORES, np_, D), jnp.float32),
        scratch_types=[
            pltpu.VMEM((k, CHUNK), jnp.int32),
            pltpu.VMEM((CHUNK, D), jnp.float32),
            pltpu.VMEM((rps, D), jnp.float32),
            pltpu.VMEM_SHARED((np_, D), jnp.float32),
        ],
    )
    def deg_kernel(dst_hbm, out_hbm, dst_v, ones_v, zbuf_v, acc):
        cid = lax.axis_index("c")
        sid = lax.axis_index("s")
        wid = cid * NSUB + sid

        @pl.loop(0, rps)
        def _(r):
            zbuf_v[r, :] = jnp.zeros((D,), jnp.float32)

        @pl.loop(0, CHUNK)
        def _(r):
            ones_v[r, :] = jnp.ones((D,), jnp.float32)

        pltpu.sync_copy(zbuf_v, acc.at[pl.ds(sid * rps, rps)])
        plsc.subcore_barrier()

        pltpu.sync_copy(dst_hbm.at[wid], dst_v)

        @pl.loop(0, k)
        def _(j):
            pltpu.sync_copy(ones_v, acc.at[dst_v.at[j]], add=True)

        plsc.subcore_barrier()
        pltpu.sync_copy(
            acc.at[pl.ds(sid * rps, rps)],
            out_hbm.at[cid, pl.ds(sid * rps, rps)],
        )

    return deg_kernel(dstp)


@functools.partial(jax.jit, static_argnames=("n", "k"))
def _gather_scatter_pass(y, srcp, dstp, *, n, k):
    np_ = _pad_rows(n)
    rps = np_ // NSUB

    stage = n // NSUB
    tail = n - NSUB * stage

    @functools.partial(
        pl.kernel,
        mesh=_mesh,
        compiler_params=_sc_params,
        out_type=jax.ShapeDtypeStruct((NCORES, np_, D), jnp.float32),
        scratch_types=[
            pltpu.VMEM((k, CHUNK), jnp.int32),
            pltpu.VMEM((k, CHUNK), jnp.int32),
            pltpu.VMEM((CHUNK, D), jnp.float32),
            pltpu.VMEM((CHUNK, D), jnp.float32),
            pltpu.VMEM((rps, D), jnp.float32),
            pltpu.VMEM_SHARED((n, D), jnp.float32),
            pltpu.VMEM_SHARED((np_, D), jnp.float32),
            pltpu.SemaphoreType.DMA,
            pltpu.SemaphoreType.DMA,
        ],
    )
    def scat_kernel(y_hbm, src_hbm, dst_hbm, out_hbm, src_v, dst_v,
                    rows0_v, rows1_v, zbuf_v, y_spm, acc, sem0, sem1):
        cid = lax.axis_index("c")
        sid = lax.axis_index("s")
        wid = cid * NSUB + sid

        pltpu.sync_copy(y_hbm.at[pl.ds(sid * stage, stage)],
                        y_spm.at[pl.ds(sid * stage, stage)])
        if tail:
            @pl.when(sid == 0)
            def _():
                pltpu.sync_copy(y_hbm.at[pl.ds(NSUB * stage, tail)],
                                y_spm.at[pl.ds(NSUB * stage, tail)])

        @pl.loop(0, rps)
        def _(r):
            zbuf_v[r, :] = jnp.zeros((D,), jnp.float32)

        pltpu.sync_copy(zbuf_v, acc.at[pl.ds(sid * rps, rps)])
        plsc.subcore_barrier()

        pltpu.sync_copy(src_hbm.at[wid], src_v)
        pltpu.sync_copy(dst_hbm.at[wid], dst_v)

        pltpu.async_copy(y_spm.at[src_v.at[0]], rows0_v, sem0).wait()

        @pl.loop(0, (k - 1) // 2)
        def _(jj):
            j = jj * 2
            pltpu.async_copy(y_spm.at[src_v.at[j + 1]], rows1_v, sem1)
            pltpu.sync_copy(rows0_v, acc.at[dst_v.at[j]], add=True)
            pltpu.make_async_copy(y_spm.at[src_v.at[j + 1]], rows1_v, sem1).wait()
            pltpu.async_copy(y_spm.at[src_v.at[j + 2]], rows0_v, sem0)
            pltpu.sync_copy(rows1_v, acc.at[dst_v.at[j + 1]], add=True)
            pltpu.make_async_copy(y_spm.at[src_v.at[j + 2]], rows0_v, sem0).wait()

        pltpu.sync_copy(rows0_v, acc.at[dst_v.at[k - 1]], add=True)

        plsc.subcore_barrier()
        pltpu.sync_copy(
            acc.at[pl.ds(sid * rps, rps)],
            out_hbm.at[cid, pl.ds(sid * rps, rps)],
        )

    return scat_kernel(y, srcp, dstp)


@functools.partial(jax.jit, static_argnames=("n", "k"))
def _fused_pass1(xw, degp, srcp, dstp, *, n, k):
    np_ = _pad_rows(n)
    rps = np_ // NSUB
    stage = n // NSUB
    tail = n - NSUB * stage

    @functools.partial(
        pl.kernel,
        mesh=_mesh,
        compiler_params=_sc_params,
        out_type=(
            jax.ShapeDtypeStruct((NCORES, np_, D), jnp.float32),
            jax.ShapeDtypeStruct((n, D), jnp.float32),
            jax.ShapeDtypeStruct((n, D), jnp.float32),
        ),
        scratch_types=[
            pltpu.VMEM((k, CHUNK), jnp.int32),
            pltpu.VMEM((k, CHUNK), jnp.int32),
            pltpu.VMEM((CHUNK, D), jnp.float32),
            pltpu.VMEM((CHUNK, D), jnp.float32),
            pltpu.VMEM((rps, D), jnp.float32),
            pltpu.VMEM((stage + (NSUB * 8), D), jnp.float32),
            pltpu.VMEM((stage + (NSUB * 8), D), jnp.float32),
            pltpu.VMEM((stage + (NSUB * 8), D), jnp.float32),
            pltpu.VMEM_SHARED((n, D), jnp.float32),
            pltpu.VMEM_SHARED((np_, D), jnp.float32),
            pltpu.SemaphoreType.DMA,
            pltpu.SemaphoreType.DMA,
        ],
    )
    def fused_kernel(xw_hbm, degp_hbm, src_hbm, dst_hbm,
                     out_hbm, dis_hbm, dinv_hbm,
                     src_v, dst_v, rows0_v, rows1_v, zbuf_v,
                     d0buf, d1buf, xwbuf, y_spm, acc, sem0, sem1):
        cid = lax.axis_index("c")
        sid = lax.axis_index("s")
        wid = cid * NSUB + sid

        @pl.loop(0, rps)
        def _(r):
            zbuf_v[r, :] = jnp.zeros((D,), jnp.float32)

        pltpu.sync_copy(zbuf_v, acc.at[pl.ds(sid * rps, rps)])

        base = sid * stage
        pltpu.sync_copy(degp_hbm.at[0, pl.ds(base, stage)],
                        d0buf.at[pl.ds(0, stage)])
        pltpu.sync_copy(degp_hbm.at[1, pl.ds(base, stage)],
                        d1buf.at[pl.ds(0, stage)])
        pltpu.sync_copy(xw_hbm.at[pl.ds(base, stage)],
                        xwbuf.at[pl.ds(0, stage)])
        if tail:
            @pl.when(sid == 0)
            def _():
                pltpu.sync_copy(degp_hbm.at[0, pl.ds(NSUB * stage, tail)],
                                d0buf.at[pl.ds(stage, tail)])
                pltpu.sync_copy(degp_hbm.at[1, pl.ds(NSUB * stage, tail)],
                                d1buf.at[pl.ds(stage, tail)])
                pltpu.sync_copy(xw_hbm.at[pl.ds(NSUB * stage, tail)],
                                xwbuf.at[pl.ds(stage, tail)])

        onev = jnp.full((D,), 1.0, jnp.float32)
        halfv = jnp.full((D,), 0.5, jnp.float32)
        threehalfv = jnp.full((D,), 1.5, jnp.float32)
        magicv = jnp.full((D,), 0x5F3759DF, jnp.int32)
        shiftv = jnp.full((D,), 1, jnp.int32)

        def finalize_row(r):
            deg = d0buf[r, :] + d1buf[r, :] + onev
            i = lax.bitcast_convert_type(deg, jnp.int32)
            i = magicv - lax.shift_right_arithmetic(i, shiftv)
            ys = lax.bitcast_convert_type(i, jnp.float32)
            for _ in range(3):
                ys = ys * (threehalfv - halfv * deg * ys * ys)
            d0buf[r, :] = xwbuf[r, :] * ys
            d1buf[r, :] = ys
            xwbuf[r, :] = onev / deg

        @pl.loop(0, stage)
        def _(r):
            finalize_row(r)
        if tail:
            @pl.when(sid == 0)
            def _():
                @pl.loop(stage, stage + tail)
                def _(r):
                    finalize_row(r)

        pltpu.sync_copy(d0buf.at[pl.ds(0, stage)],
                        y_spm.at[pl.ds(base, stage)])

        @pl.when(cid == 0)
        def _():
            pltpu.sync_copy(d1buf.at[pl.ds(0, stage)],
                            dis_hbm.at[pl.ds(base, stage)])
            pltpu.sync_copy(xwbuf.at[pl.ds(0, stage)],
                            dinv_hbm.at[pl.ds(base, stage)])
        if tail:
            @pl.when(sid == 0)
            def _():
                pltpu.sync_copy(d0buf.at[pl.ds(stage, tail)],
                                y_spm.at[pl.ds(NSUB * stage, tail)])

                @pl.when(cid == 0)
                def _():
                    pltpu.sync_copy(d1buf.at[pl.ds(stage, tail)],
                                    dis_hbm.at[pl.ds(NSUB * stage, tail)])
                    pltpu.sync_copy(xwbuf.at[pl.ds(stage, tail)],
                                    dinv_hbm.at[pl.ds(NSUB * stage, tail)])

        plsc.subcore_barrier()

        pltpu.sync_copy(src_hbm.at[wid], src_v)
        pltpu.sync_copy(dst_hbm.at[wid], dst_v)

        pltpu.async_copy(y_spm.at[src_v.at[0]], rows0_v, sem0).wait()

        @pl.loop(0, (k - 1) // 2)
        def _(jj):
            j = jj * 2
            pltpu.async_copy(y_spm.at[src_v.at[j + 1]], rows1_v, sem1)
            pltpu.sync_copy(rows0_v, acc.at[dst_v.at[j]], add=True)
            pltpu.make_async_copy(y_spm.at[src_v.at[j + 1]], rows1_v, sem1).wait()
            pltpu.async_copy(y_spm.at[src_v.at[j + 2]], rows0_v, sem0)
            pltpu.sync_copy(rows1_v, acc.at[dst_v.at[j + 1]], add=True)
            pltpu.make_async_copy(y_spm.at[src_v.at[j + 2]], rows0_v, sem0).wait()

        pltpu.sync_copy(rows0_v, acc.at[dst_v.at[k - 1]], add=True)

        plsc.subcore_barrier()
        pltpu.sync_copy(
            acc.at[pl.ds(sid * rps, rps)],
            out_hbm.at[cid, pl.ds(sid * rps, rps)],
        )

    return fused_kernel(xw, degp, srcp, dstp)


def _mm_body(x_ref, w_ref, o_ref):
    o_ref[...] = jnp.dot(x_ref[...], w_ref[...],
                         preferred_element_type=jnp.float32)


def _layer1_body(s0_ref, s1_ref, xw_ref, dis_ref, dinv_ref, b_ref, w2_ref,
                 y2_ref, hw2_ref):
    dis = dis_ref[...]
    pre = dis * (s0_ref[...] + s1_ref[...]) + xw_ref[...] * dinv_ref[...] \
        + b_ref[...]
    h = jnp.maximum(pre, 0.0)
    hw2 = jnp.dot(h, w2_ref[...], preferred_element_type=jnp.float32)
    hw2_ref[...] = hw2
    y2_ref[...] = hw2 * dis


def _layer2_body(s0_ref, s1_ref, hw2_ref, dis_ref, dinv_ref, b_ref, o_ref):
    o_ref[...] = dis_ref[...] * (s0_ref[...] + s1_ref[...]) \
        + hw2_ref[...] * dinv_ref[...] + b_ref[...]


def kernel(x, edge_index, W1, b1, W2, b2):
    n = x.shape[0]
    e = edge_index.shape[1]
    np_ = _pad_rows(n)
    k = -(-e // (NTILES * CHUNK))
    if k % 2 == 0:
        k += 1
    per_tile = k * CHUNK
    ep = NTILES * per_tile

    src = edge_index[0]
    dst = edge_index[1]
    pad = ep - e
    srcp = jnp.concatenate([src, jnp.zeros((pad,), jnp.int32)])
    dstp = jnp.concatenate([dst, jnp.full((pad,), n, jnp.int32)])
    srcp = srcp.reshape(NTILES, k, CHUNK)
    dstp = dstp.reshape(NTILES, k, CHUNK)

    fshape = jax.ShapeDtypeStruct((n, D), jnp.float32)

    degp = _deg_pass(dstp, n=n, k=k)
    xw1 = pl.pallas_call(_mm_body, out_shape=fshape)(x, W1)

    s1p, dis, dinv = _fused_pass1(xw1, degp, srcp, dstp, n=n, k=k)

    y2, hw2 = pl.pallas_call(
        _layer1_body, out_shape=(fshape, fshape)
    )(s1p[0, :n], s1p[1, :n], xw1, dis, dinv, b1.reshape(1, D), W2)

    s2p = _gather_scatter_pass(y2, srcp, dstp, n=n, k=k)

    out = pl.pallas_call(
        _layer2_body, out_shape=fshape
    )(s2p[0, :n], s2p[1, :n], hw2, dis, dinv, b2.reshape(1, D))
    return out


# speedup vs baseline: 55.1333x; 1.1109x over previous
import functools

import jax
import jax.numpy as jnp
from jax import lax
from jax.experimental import pallas as pl
from jax.experimental.pallas import tpu as pltpu
from jax.experimental.pallas import tpu_sc as plsc

D = 16
CHUNK = 128
NCORES = 2
NSUB = 16
NTILES = NCORES * NSUB

_mesh = plsc.VectorSubcoreMesh(core_axis_name="c", subcore_axis_name="s")
_sc_params = pltpu.CompilerParams(use_tc_tiling_on_sc=False)
_sc_params_nl = pltpu.CompilerParams(use_tc_tiling_on_sc=False,
                                     needs_layout_passes=False)


def _pad_rows(n):
    return ((n + 1 + NSUB * 8 - 1) // (NSUB * 8)) * (NSUB * 8)


@functools.partial(jax.jit, static_argnames=("n", "k", "chunk"))
def _deg_pass(edge_r, *, n, k, chunk):
    np_ = _pad_rows(n)
    rps = np_ // NSUB

    @functools.partial(
        pl.kernel,
        mesh=_mesh,
        compiler_params=_sc_params,
        out_type=jax.ShapeDtypeStruct((NCORES, np_, D), jnp.float32),
        scratch_types=[
            pltpu.VMEM((k, chunk), jnp.int32),
            pltpu.VMEM((chunk, D), jnp.float32),
            pltpu.VMEM((rps, D), jnp.float32),
            pltpu.VMEM_SHARED((np_, D), jnp.float32),
        ],
    )
    def deg_kernel(edge_hbm, out_hbm, dst_v, ones_v, zbuf_v, acc):
        cid = lax.axis_index("c")
        sid = lax.axis_index("s")
        wid = cid * NSUB + sid

        @pl.loop(0, rps)
        def _(r):
            zbuf_v[r, :] = jnp.zeros((D,), jnp.float32)

        @pl.loop(0, chunk)
        def _(r):
            ones_v[r, :] = jnp.ones((D,), jnp.float32)

        pltpu.sync_copy(zbuf_v, acc.at[pl.ds(sid * rps, rps)])
        plsc.subcore_barrier()

        pltpu.sync_copy(edge_hbm.at[1, wid], dst_v)

        @pl.loop(0, k)
        def _(j):
            pltpu.sync_copy(ones_v, acc.at[dst_v.at[j]], add=True)

        plsc.subcore_barrier()
        pltpu.sync_copy(
            acc.at[pl.ds(sid * rps, rps)],
            out_hbm.at[cid, pl.ds(sid * rps, rps)],
        )

    return deg_kernel(edge_r)


@functools.partial(jax.jit, static_argnames=("n", "k", "chunk"))
def _gather_scatter_pass(y, edge_r, *, n, k, chunk):
    np_ = _pad_rows(n)
    rps = np_ // NSUB

    stage = n // NSUB
    tail = n - NSUB * stage

    @functools.partial(
        pl.kernel,
        mesh=_mesh,
        compiler_params=_sc_params,
        out_type=jax.ShapeDtypeStruct((NCORES, np_, D), jnp.float32),
        scratch_types=[
            pltpu.VMEM((k, chunk), jnp.int32),
            pltpu.VMEM((k, chunk), jnp.int32),
            pltpu.VMEM((chunk, D), jnp.float32),
            pltpu.VMEM((chunk, D), jnp.float32),
            pltpu.VMEM((rps, D), jnp.float32),
            pltpu.VMEM_SHARED((n, D), jnp.float32),
            pltpu.VMEM_SHARED((np_, D), jnp.float32),
            pltpu.SemaphoreType.DMA,
            pltpu.SemaphoreType.DMA,
        ],
    )
    def scat_kernel(y_hbm, edge_hbm, out_hbm, src_v, dst_v,
                    rows0_v, rows1_v, zbuf_v, y_spm, acc, sem0, sem1):
        cid = lax.axis_index("c")
        sid = lax.axis_index("s")
        wid = cid * NSUB + sid

        pltpu.sync_copy(y_hbm.at[pl.ds(sid * stage, stage)],
                        y_spm.at[pl.ds(sid * stage, stage)])
        if tail:
            @pl.when(sid == 0)
            def _():
                pltpu.sync_copy(y_hbm.at[pl.ds(NSUB * stage, tail)],
                                y_spm.at[pl.ds(NSUB * stage, tail)])

        @pl.loop(0, rps)
        def _(r):
            zbuf_v[r, :] = jnp.zeros((D,), jnp.float32)

        pltpu.sync_copy(zbuf_v, acc.at[pl.ds(sid * rps, rps)])
        plsc.subcore_barrier()

        pltpu.sync_copy(edge_hbm.at[0, wid], src_v)
        pltpu.sync_copy(edge_hbm.at[1, wid], dst_v)

        pltpu.async_copy(y_spm.at[src_v.at[0]], rows0_v, sem0).wait()

        @pl.loop(0, (k - 1) // 2)
        def _(jj):
            j = jj * 2
            pltpu.async_copy(y_spm.at[src_v.at[j + 1]], rows1_v, sem1)
            pltpu.sync_copy(rows0_v, acc.at[dst_v.at[j]], add=True)
            pltpu.make_async_copy(y_spm.at[src_v.at[j + 1]], rows1_v, sem1).wait()
            pltpu.async_copy(y_spm.at[src_v.at[j + 2]], rows0_v, sem0)
            pltpu.sync_copy(rows1_v, acc.at[dst_v.at[j + 1]], add=True)
            pltpu.make_async_copy(y_spm.at[src_v.at[j + 2]], rows0_v, sem0).wait()

        pltpu.sync_copy(rows0_v, acc.at[dst_v.at[k - 1]], add=True)

        plsc.subcore_barrier()
        pltpu.sync_copy(
            acc.at[pl.ds(sid * rps, rps)],
            out_hbm.at[cid, pl.ds(sid * rps, rps)],
        )

    return scat_kernel(y, edge_r)


@functools.partial(jax.jit, static_argnames=("n", "k", "chunk"))
def _fused_pass1(xw, degp, edge_r, *, n, k, chunk):
    np_ = _pad_rows(n)
    rps = np_ // NSUB
    stage = n // NSUB
    tail = n - NSUB * stage

    @functools.partial(
        pl.kernel,
        mesh=_mesh,
        compiler_params=_sc_params,
        out_type=(
            jax.ShapeDtypeStruct((NCORES, np_, D), jnp.float32),
            jax.ShapeDtypeStruct((n, D), jnp.float32),
            jax.ShapeDtypeStruct((n, D), jnp.float32),
        ),
        scratch_types=[
            pltpu.VMEM((k, chunk), jnp.int32),
            pltpu.VMEM((k, chunk), jnp.int32),
            pltpu.VMEM((chunk, D), jnp.float32),
            pltpu.VMEM((chunk, D), jnp.float32),
            pltpu.VMEM((rps, D), jnp.float32),
            pltpu.VMEM((stage + (NSUB * 8), D), jnp.float32),
            pltpu.VMEM((stage + (NSUB * 8), D), jnp.float32),
            pltpu.VMEM((stage + (NSUB * 8), D), jnp.float32),
            pltpu.VMEM_SHARED((n, D), jnp.float32),
            pltpu.VMEM_SHARED((np_, D), jnp.float32),
            pltpu.SemaphoreType.DMA,
            pltpu.SemaphoreType.DMA,
        ],
    )
    def fused_kernel(xw_hbm, degp_hbm, edge_hbm,
                     out_hbm, dis_hbm, dinv_hbm,
                     src_v, dst_v, rows0_v, rows1_v, zbuf_v,
                     d0buf, d1buf, xwbuf, y_spm, acc, sem0, sem1):
        cid = lax.axis_index("c")
        sid = lax.axis_index("s")
        wid = cid * NSUB + sid

        @pl.loop(0, rps)
        def _(r):
            zbuf_v[r, :] = jnp.zeros((D,), jnp.float32)

        pltpu.sync_copy(zbuf_v, acc.at[pl.ds(sid * rps, rps)])

        base = sid * stage
        pltpu.sync_copy(degp_hbm.at[0, pl.ds(base, stage)],
                        d0buf.at[pl.ds(0, stage)])
        pltpu.sync_copy(degp_hbm.at[1, pl.ds(base, stage)],
                        d1buf.at[pl.ds(0, stage)])
        pltpu.sync_copy(xw_hbm.at[pl.ds(base, stage)],
                        xwbuf.at[pl.ds(0, stage)])
        if tail:
            @pl.when(sid == 0)
            def _():
                pltpu.sync_copy(degp_hbm.at[0, pl.ds(NSUB * stage, tail)],
                                d0buf.at[pl.ds(stage, tail)])
                pltpu.sync_copy(degp_hbm.at[1, pl.ds(NSUB * stage, tail)],
                                d1buf.at[pl.ds(stage, tail)])
                pltpu.sync_copy(xw_hbm.at[pl.ds(NSUB * stage, tail)],
                                xwbuf.at[pl.ds(stage, tail)])

        onev = jnp.full((D,), 1.0, jnp.float32)
        halfv = jnp.full((D,), 0.5, jnp.float32)
        threehalfv = jnp.full((D,), 1.5, jnp.float32)
        magicv = jnp.full((D,), 0x5F3759DF, jnp.int32)
        shiftv = jnp.full((D,), 1, jnp.int32)

        def finalize_row(r):
            deg = d0buf[r, :] + d1buf[r, :] + onev
            i = lax.bitcast_convert_type(deg, jnp.int32)
            i = magicv - lax.shift_right_arithmetic(i, shiftv)
            ys = lax.bitcast_convert_type(i, jnp.float32)
            for _ in range(3):
                ys = ys * (threehalfv - halfv * deg * ys * ys)
            d0buf[r, :] = xwbuf[r, :] * ys
            d1buf[r, :] = ys
            xwbuf[r, :] = onev / deg

        @pl.loop(0, stage)
        def _(r):
            finalize_row(r)
        if tail:
            @pl.when(sid == 0)
            def _():
                @pl.loop(stage, stage + tail)
                def _(r):
                    finalize_row(r)

        pltpu.sync_copy(d0buf.at[pl.ds(0, stage)],
                        y_spm.at[pl.ds(base, stage)])

        @pl.when(cid == 0)
        def _():
            pltpu.sync_copy(d1buf.at[pl.ds(0, stage)],
                            dis_hbm.at[pl.ds(base, stage)])
            pltpu.sync_copy(xwbuf.at[pl.ds(0, stage)],
                            dinv_hbm.at[pl.ds(base, stage)])
        if tail:
            @pl.when(sid == 0)
            def _():
                pltpu.sync_copy(d0buf.at[pl.ds(stage, tail)],
                                y_spm.at[pl.ds(NSUB * stage, tail)])

                @pl.when(cid == 0)
                def _():
                    pltpu.sync_copy(d1buf.at[pl.ds(stage, tail)],
                                    dis_hbm.at[pl.ds(NSUB * stage, tail)])
                    pltpu.sync_copy(xwbuf.at[pl.ds(stage, tail)],
                                    dinv_hbm.at[pl.ds(NSUB * stage, tail)])

        plsc.subcore_barrier()

        pltpu.sync_copy(edge_hbm.at[0, wid], src_v)
        pltpu.sync_copy(edge_hbm.at[1, wid], dst_v)

        pltpu.async_copy(y_spm.at[src_v.at[0]], rows0_v, sem0).wait()

        @pl.loop(0, (k - 1) // 2)
        def _(jj):
            j = jj * 2
            pltpu.async_copy(y_spm.at[src_v.at[j + 1]], rows1_v, sem1)
            pltpu.sync_copy(rows0_v, acc.at[dst_v.at[j]], add=True)
            pltpu.make_async_copy(y_spm.at[src_v.at[j + 1]], rows1_v, sem1).wait()
            pltpu.async_copy(y_spm.at[src_v.at[j + 2]], rows0_v, sem0)
            pltpu.sync_copy(rows1_v, acc.at[dst_v.at[j + 1]], add=True)
            pltpu.make_async_copy(y_spm.at[src_v.at[j + 2]], rows0_v, sem0).wait()

        pltpu.sync_copy(rows0_v, acc.at[dst_v.at[k - 1]], add=True)

        plsc.subcore_barrier()
        pltpu.sync_copy(
            acc.at[pl.ds(sid * rps, rps)],
            out_hbm.at[cid, pl.ds(sid * rps, rps)],
        )

    return fused_kernel(xw, degp, edge_r)


def _mm_body(x_ref, w_ref, o_ref):
    o_ref[...] = jnp.dot(x_ref[...], w_ref[...],
                         preferred_element_type=jnp.float32)


def _make_layer1_body(n):
    def _layer1_body(sp_ref, xw_ref, dis_ref, dinv_ref, b_ref, w2_ref,
                     y2_ref, hw2_ref):
        dis = dis_ref[...]
        s = sp_ref[0, :n, :] + sp_ref[1, :n, :]
        pre = dis * s + xw_ref[...] * dinv_ref[...] + b_ref[...]
        h = jnp.maximum(pre, 0.0)
        hw2 = jnp.dot(h, w2_ref[...], preferred_element_type=jnp.float32)
        hw2_ref[...] = hw2
        y2_ref[...] = hw2 * dis
    return _layer1_body


def _make_layer2_body(n):
    def _layer2_body(sp_ref, hw2_ref, dis_ref, dinv_ref, b_ref, o_ref):
        s = sp_ref[0, :n, :] + sp_ref[1, :n, :]
        o_ref[...] = dis_ref[...] * s + hw2_ref[...] * dinv_ref[...] \
            + b_ref[...]
    return _layer2_body


def _pick_chunk(ept):
    for c in range(128, 7, -8):
        if ept % c == 0 and (ept // c) % 2 == 1 and ept // c >= 3:
            return c
    return None


def kernel(x, edge_index, W1, b1, W2, b2):
    n = x.shape[0]
    e = edge_index.shape[1]
    np_ = _pad_rows(n)

    chunk = _pick_chunk(e // NTILES) if e % NTILES == 0 else None
    if chunk is not None:
        k = e // (NTILES * chunk)
        edge_r = edge_index.reshape(2, NTILES, k, chunk)
    else:
        chunk = CHUNK
        k = -(-e // (NTILES * chunk))
        if k % 2 == 0:
            k += 1
        pad = NTILES * k * chunk - e
        srcp = jnp.concatenate([edge_index[0], jnp.zeros((pad,), jnp.int32)])
        dstp = jnp.concatenate([edge_index[1], jnp.full((pad,), n, jnp.int32)])
        edge_r = jnp.stack([srcp, dstp]).reshape(2, NTILES, k, chunk)

    fshape = jax.ShapeDtypeStruct((n, D), jnp.float32)

    degp = _deg_pass(edge_r, n=n, k=k, chunk=chunk)
    xw1 = pl.pallas_call(_mm_body, out_shape=fshape)(x, W1)

    s1p, dis, dinv = _fused_pass1(xw1, degp, edge_r, n=n, k=k, chunk=chunk)

    y2, hw2 = pl.pallas_call(
        _make_layer1_body(n), out_shape=(fshape, fshape)
    )(s1p, xw1, dis, dinv, b1.reshape(1, D), W2)

    s2p = _gather_scatter_pass(y2, edge_r, n=n, k=k, chunk=chunk)

    out = pl.pallas_call(
        _make_layer2_body(n), out_shape=fshape
    )(s2p, hw2, dis, dinv, b2.reshape(1, D))
    return out


# speedup vs baseline: 57.8886x; 1.0500x over previous
import functools

import jax
import jax.numpy as jnp
from jax import lax
from jax.experimental import pallas as pl
from jax.experimental.pallas import tpu as pltpu
from jax.experimental.pallas import tpu_sc as plsc

D = 16
CHUNK = 128
NCORES = 2
NSUB = 16
NTILES = NCORES * NSUB

_mesh = plsc.VectorSubcoreMesh(core_axis_name="c", subcore_axis_name="s")
_sc_params = pltpu.CompilerParams(use_tc_tiling_on_sc=False)
_sc_params_nl = pltpu.CompilerParams(use_tc_tiling_on_sc=False,
                                     needs_layout_passes=False)


def _pad_rows(n):
    return ((n + NSUB + NSUB * 8 - 1) // (NSUB * 8)) * (NSUB * 8)


@functools.partial(jax.jit, static_argnames=("n", "k", "chunk"))
def _deg_pass(edge_r, *, n, k, chunk):
    np_ = _pad_rows(n)
    rps = np_ // NSUB

    @functools.partial(
        pl.kernel,
        mesh=_mesh,
        compiler_params=_sc_params,
        out_type=jax.ShapeDtypeStruct((NCORES, np_, D), jnp.float32),
        scratch_types=[
            pltpu.VMEM((k, chunk), jnp.int32),
            pltpu.VMEM((chunk, D), jnp.float32),
            pltpu.VMEM((rps, D), jnp.float32),
            pltpu.VMEM_SHARED((np_, D), jnp.float32),
        ],
    )
    def deg_kernel(edge_hbm, out_hbm, dst_v, ones_v, zbuf_v, acc):
        cid = lax.axis_index("c")
        sid = lax.axis_index("s")
        wid = cid * NSUB + sid

        @pl.loop(0, rps)
        def _(r):
            zbuf_v[r, :] = jnp.zeros((D,), jnp.float32)

        @pl.loop(0, chunk)
        def _(r):
            ones_v[r, :] = jnp.ones((D,), jnp.float32)

        pltpu.sync_copy(zbuf_v, acc.at[pl.ds(sid * rps, rps)])
        plsc.subcore_barrier()

        pltpu.sync_copy(edge_hbm.at[1, wid], dst_v)

        @pl.loop(0, k)
        def _(j):
            pltpu.sync_copy(ones_v, acc.at[dst_v.at[j]], add=True)

        plsc.subcore_barrier()
        pltpu.sync_copy(
            acc.at[pl.ds(sid * rps, rps)],
            out_hbm.at[cid, pl.ds(sid * rps, rps)],
        )

    return deg_kernel(edge_r)


@functools.partial(jax.jit, static_argnames=("n", "k", "chunk"))
def _gather_scatter_pass(y, edge_r, *, n, k, chunk):
    np_ = _pad_rows(n)
    rps = np_ // NSUB

    stage = n // NSUB
    tail = n - NSUB * stage

    @functools.partial(
        pl.kernel,
        mesh=_mesh,
        compiler_params=_sc_params,
        out_type=jax.ShapeDtypeStruct((NCORES, np_, D), jnp.float32),
        scratch_types=[
            pltpu.VMEM((k, chunk), jnp.int32),
            pltpu.VMEM((k, chunk), jnp.int32),
            pltpu.VMEM((chunk, D), jnp.float32),
            pltpu.VMEM((chunk, D), jnp.float32),
            pltpu.VMEM((rps, D), jnp.float32),
            pltpu.VMEM_SHARED((n, D), jnp.float32),
            pltpu.VMEM_SHARED((np_, D), jnp.float32),
            pltpu.SemaphoreType.DMA,
            pltpu.SemaphoreType.DMA,
        ],
    )
    def scat_kernel(y_hbm, edge_hbm, out_hbm, src_v, dst_v,
                    rows0_v, rows1_v, zbuf_v, y_spm, acc, sem0, sem1):
        cid = lax.axis_index("c")
        sid = lax.axis_index("s")
        wid = cid * NSUB + sid

        pltpu.sync_copy(y_hbm.at[pl.ds(sid * stage, stage)],
                        y_spm.at[pl.ds(sid * stage, stage)])
        if tail:
            @pl.when(sid == 0)
            def _():
                pltpu.sync_copy(y_hbm.at[pl.ds(NSUB * stage, tail)],
                                y_spm.at[pl.ds(NSUB * stage, tail)])

        @pl.loop(0, rps)
        def _(r):
            zbuf_v[r, :] = jnp.zeros((D,), jnp.float32)

        pltpu.sync_copy(zbuf_v, acc.at[pl.ds(sid * rps, rps)])
        plsc.subcore_barrier()

        pltpu.sync_copy(edge_hbm.at[0, wid], src_v)
        pltpu.sync_copy(edge_hbm.at[1, wid], dst_v)

        pltpu.async_copy(y_spm.at[src_v.at[0]], rows0_v, sem0).wait()

        @pl.loop(0, (k - 1) // 2)
        def _(jj):
            j = jj * 2
            pltpu.async_copy(y_spm.at[src_v.at[j + 1]], rows1_v, sem1)
            pltpu.sync_copy(rows0_v, acc.at[dst_v.at[j]], add=True)
            pltpu.make_async_copy(y_spm.at[src_v.at[j + 1]], rows1_v, sem1).wait()
            pltpu.async_copy(y_spm.at[src_v.at[j + 2]], rows0_v, sem0)
            pltpu.sync_copy(rows1_v, acc.at[dst_v.at[j + 1]], add=True)
            pltpu.make_async_copy(y_spm.at[src_v.at[j + 2]], rows0_v, sem0).wait()

        pltpu.sync_copy(rows0_v, acc.at[dst_v.at[k - 1]], add=True)

        plsc.subcore_barrier()
        pltpu.sync_copy(
            acc.at[pl.ds(sid * rps, rps)],
            out_hbm.at[cid, pl.ds(sid * rps, rps)],
        )

    return scat_kernel(y, edge_r)


@functools.partial(jax.jit, static_argnames=("n", "k", "chunk"))
def _fused_pass1(xw, degp, edge_r, *, n, k, chunk):
    np_ = _pad_rows(n)
    rps = np_ // NSUB
    stage = n // NSUB
    tail = n - NSUB * stage
    degstage = -(-stage // D) * D

    @functools.partial(
        pl.kernel,
        mesh=_mesh,
        compiler_params=_sc_params_nl,
        out_type=(
            jax.ShapeDtypeStruct((NCORES, np_, D), jnp.float32),
            jax.ShapeDtypeStruct((n, D), jnp.float32),
            jax.ShapeDtypeStruct((n, D), jnp.float32),
        ),
        scratch_types=[
            pltpu.VMEM((k, chunk), jnp.int32),
            pltpu.VMEM((k, chunk), jnp.int32),
            pltpu.VMEM((chunk, D), jnp.float32),
            pltpu.VMEM((chunk, D), jnp.float32),
            pltpu.VMEM((rps, D), jnp.float32),
            pltpu.VMEM((degstage, D), jnp.float32),
            pltpu.VMEM((degstage, D), jnp.float32),
            pltpu.VMEM((stage + (NSUB * 8), D), jnp.float32),
            pltpu.VMEM((degstage,), jnp.float32),
            pltpu.VMEM_SHARED((n, D), jnp.float32),
            pltpu.VMEM_SHARED((np_, D), jnp.float32),
            pltpu.SemaphoreType.DMA,
            pltpu.SemaphoreType.DMA,
        ],
    )
    def fused_kernel(xw_hbm, degp_hbm, edge_hbm,
                     out_hbm, dis_hbm, dinv_hbm,
                     src_v, dst_v, rows0_v, rows1_v, zbuf_v,
                     d0buf, d1buf, xwbuf, dis_pack, y_spm, acc, sem0, sem1):
        cid = lax.axis_index("c")
        sid = lax.axis_index("s")
        wid = cid * NSUB + sid

        iotav = lax.iota(jnp.int32, D)
        zerovf = jnp.zeros((D,), jnp.float32)

        @pl.loop(0, rps)
        def _(r):
            plsc.store_scatter(zbuf_v, [jnp.full((D,), r, jnp.int32), iotav],
                               zerovf)

        pltpu.sync_copy(zbuf_v, acc.at[pl.ds(sid * rps, rps)])

        base = sid * stage
        pltpu.sync_copy(degp_hbm.at[0, pl.ds(base, degstage)],
                        d0buf.at[pl.ds(0, degstage)])
        pltpu.sync_copy(degp_hbm.at[1, pl.ds(base, degstage)],
                        d1buf.at[pl.ds(0, degstage)])
        pltpu.sync_copy(xw_hbm.at[pl.ds(base, stage)],
                        xwbuf.at[pl.ds(0, stage)])

        onev = jnp.full((D,), 1.0, jnp.float32)
        halfv = jnp.full((D,), 0.5, jnp.float32)
        threehalfv = jnp.full((D,), 1.5, jnp.float32)
        magicv = jnp.full((D,), 0x5F3759DF, jnp.int32)
        shiftv = jnp.full((D,), 1, jnp.int32)
        zerov = jnp.zeros((D,), jnp.int32)

        def row_load(buf, r):
            return plsc.load_gather(buf, [jnp.full((D,), r, jnp.int32), iotav])

        def row_store(buf, r, x):
            plsc.store_scatter(buf, [jnp.full((D,), r, jnp.int32), iotav], x)

        def newton(deg):
            i = lax.bitcast_convert_type(deg, jnp.int32)
            i = magicv - lax.shift_right_arithmetic(i, shiftv)
            ys = lax.bitcast_convert_type(i, jnp.float32)
            for _ in range(3):
                ys = ys * (threehalfv - halfv * deg * ys * ys)
            return ys

        @pl.loop(0, degstage // D)
        def _(v):
            rows = iotav + v * D
            g0 = plsc.load_gather(d0buf, [rows, zerov])
            g1 = plsc.load_gather(d1buf, [rows, zerov])
            dis_pack[pl.ds(v * D, D)] = newton(g0 + g1 + onev)

        @pl.loop(0, stage)
        def _(r):
            splat = plsc.load_gather(dis_pack, [jnp.full((D,), r, jnp.int32)])
            row_store(d0buf, r, row_load(xwbuf, r) * splat)
            row_store(d1buf, r, splat)
            row_store(xwbuf, r, splat * splat)

        if tail:
            @pl.when(sid == 0)
            def _():
                pltpu.sync_copy(degp_hbm.at[0, pl.ds(NSUB * stage, tail)],
                                rows0_v.at[pl.ds(0, tail)])
                pltpu.sync_copy(degp_hbm.at[1, pl.ds(NSUB * stage, tail)],
                                rows1_v.at[pl.ds(0, tail)])
                pltpu.sync_copy(xw_hbm.at[pl.ds(NSUB * stage, tail)],
                                xwbuf.at[pl.ds(stage, tail)])

                @pl.loop(0, tail)
                def _(r):
                    ys = newton(row_load(rows0_v, r) + row_load(rows1_v, r)
                                + onev)
                    row_store(rows0_v, r, row_load(xwbuf, stage + r) * ys)
                    row_store(rows1_v, r, ys)
                    row_store(xwbuf, stage + r, ys * ys)

        pltpu.sync_copy(d0buf.at[pl.ds(0, stage)],
                        y_spm.at[pl.ds(base, stage)])

        @pl.when(cid == 0)
        def _():
            pltpu.sync_copy(d1buf.at[pl.ds(0, stage)],
                            dis_hbm.at[pl.ds(base, stage)])
            pltpu.sync_copy(xwbuf.at[pl.ds(0, stage)],
                            dinv_hbm.at[pl.ds(base, stage)])
        if tail:
            @pl.when(sid == 0)
            def _():
                pltpu.sync_copy(rows0_v.at[pl.ds(0, tail)],
                                y_spm.at[pl.ds(NSUB * stage, tail)])

                @pl.when(cid == 0)
                def _():
                    pltpu.sync_copy(rows1_v.at[pl.ds(0, tail)],
                                    dis_hbm.at[pl.ds(NSUB * stage, tail)])
                    pltpu.sync_copy(xwbuf.at[pl.ds(stage, tail)],
                                    dinv_hbm.at[pl.ds(NSUB * stage, tail)])

        plsc.subcore_barrier()

        pltpu.sync_copy(edge_hbm.at[0, wid], src_v)
        pltpu.sync_copy(edge_hbm.at[1, wid], dst_v)

        pltpu.async_copy(y_spm.at[src_v.at[0]], rows0_v, sem0).wait()

        @pl.loop(0, (k - 1) // 2)
        def _(jj):
            j = jj * 2
            pltpu.async_copy(y_spm.at[src_v.at[j + 1]], rows1_v, sem1)
            pltpu.sync_copy(rows0_v, acc.at[dst_v.at[j]], add=True)
            pltpu.make_async_copy(y_spm.at[src_v.at[j + 1]], rows1_v, sem1).wait()
            pltpu.async_copy(y_spm.at[src_v.at[j + 2]], rows0_v, sem0)
            pltpu.sync_copy(rows1_v, acc.at[dst_v.at[j + 1]], add=True)
            pltpu.make_async_copy(y_spm.at[src_v.at[j + 2]], rows0_v, sem0).wait()

        pltpu.sync_copy(rows0_v, acc.at[dst_v.at[k - 1]], add=True)

        plsc.subcore_barrier()
        pltpu.sync_copy(
            acc.at[pl.ds(sid * rps, rps)],
            out_hbm.at[cid, pl.ds(sid * rps, rps)],
        )

    return fused_kernel(xw, degp, edge_r)


def _mm_body(x_ref, w_ref, o_ref):
    o_ref[...] = jnp.dot(x_ref[...], w_ref[...],
                         preferred_element_type=jnp.float32)


def _make_layer1_body(n):
    def _layer1_body(sp_ref, xw_ref, dis_ref, dinv_ref, b_ref, w2_ref,
                     y2_ref, hw2_ref):
        dis = dis_ref[...]
        s = sp_ref[0, :n, :] + sp_ref[1, :n, :]
        pre = dis * s + xw_ref[...] * dinv_ref[...] + b_ref[...]
        h = jnp.maximum(pre, 0.0)
        hw2 = jnp.dot(h, w2_ref[...], preferred_element_type=jnp.float32)
        hw2_ref[...] = hw2
        y2_ref[...] = hw2 * dis
    return _layer1_body


def _make_layer2_body(n):
    def _layer2_body(sp_ref, hw2_ref, dis_ref, dinv_ref, b_ref, o_ref):
        s = sp_ref[0, :n, :] + sp_ref[1, :n, :]
        o_ref[...] = dis_ref[...] * s + hw2_ref[...] * dinv_ref[...] \
            + b_ref[...]
    return _layer2_body


def _pick_chunk(ept):
    for c in range(128, 7, -8):
        if ept % c == 0 and (ept // c) % 2 == 1 and ept // c >= 3:
            return c
    return None


def kernel(x, edge_index, W1, b1, W2, b2):
    n = x.shape[0]
    e = edge_index.shape[1]
    np_ = _pad_rows(n)

    chunk = _pick_chunk(e // NTILES) if e % NTILES == 0 else None
    if chunk is not None:
        k = e // (NTILES * chunk)
        edge_r = edge_index.reshape(2, NTILES, k, chunk)
    else:
        chunk = CHUNK
        k = -(-e // (NTILES * chunk))
        if k % 2 == 0:
            k += 1
        pad = NTILES * k * chunk - e
        srcp = jnp.concatenate([edge_index[0], jnp.zeros((pad,), jnp.int32)])
        dstp = jnp.concatenate([edge_index[1], jnp.full((pad,), n, jnp.int32)])
        edge_r = jnp.stack([srcp, dstp]).reshape(2, NTILES, k, chunk)

    fshape = jax.ShapeDtypeStruct((n, D), jnp.float32)

    degp = _deg_pass(edge_r, n=n, k=k, chunk=chunk)
    xw1 = pl.pallas_call(_mm_body, out_shape=fshape)(x, W1)

    s1p, dis, dinv = _fused_pass1(xw1, degp, edge_r, n=n, k=k, chunk=chunk)

    y2, hw2 = pl.pallas_call(
        _make_layer1_body(n), out_shape=(fshape, fshape)
    )(s1p, xw1, dis, dinv, b1.reshape(1, D), W2)

    s2p = _gather_scatter_pass(y2, edge_r, n=n, k=k, chunk=chunk)

    out = pl.pallas_call(
        _make_layer2_body(n), out_shape=fshape
    )(s2p, hw2, dis, dinv, b2.reshape(1, D))
    return out


# speedup vs baseline: 76.9468x; 1.3292x over previous
import functools

import jax
import jax.numpy as jnp
from jax import lax
from jax.experimental import pallas as pl
from jax.experimental.pallas import tpu as pltpu
from jax.experimental.pallas import tpu_sc as plsc

D = 16
CHUNK = 128
NCORES = 2
NSUB = 16
NTILES = NCORES * NSUB

_mesh = plsc.VectorSubcoreMesh(core_axis_name="c", subcore_axis_name="s")
_sc_params = pltpu.CompilerParams(use_tc_tiling_on_sc=False)
_sc_params_nl = pltpu.CompilerParams(use_tc_tiling_on_sc=False,
                                     needs_layout_passes=False)


def _pad_rows(n):
    return ((n + NSUB + NSUB * 8 - 1) // (NSUB * 8)) * (NSUB * 8)


@functools.partial(jax.jit, static_argnames=("n", "k", "chunk"))
def _deg_pass(edge_r, *, n, k, chunk):
    np_ = _pad_rows(n)
    rps = np_ // NSUB

    @functools.partial(
        pl.kernel,
        mesh=_mesh,
        compiler_params=_sc_params,
        out_type=jax.ShapeDtypeStruct((NCORES, np_, D), jnp.float32),
        scratch_types=[
            pltpu.VMEM((k, chunk), jnp.int32),
            pltpu.VMEM((chunk, D), jnp.float32),
            pltpu.VMEM((rps, D), jnp.float32),
            pltpu.VMEM_SHARED((np_, D), jnp.float32),
        ],
    )
    def deg_kernel(edge_hbm, out_hbm, dst_v, ones_v, zbuf_v, acc):
        cid = lax.axis_index("c")
        sid = lax.axis_index("s")
        wid = cid * NSUB + sid

        @pl.loop(0, rps)
        def _(r):
            zbuf_v[r, :] = jnp.zeros((D,), jnp.float32)

        @pl.loop(0, chunk)
        def _(r):
            ones_v[r, :] = jnp.ones((D,), jnp.float32)

        pltpu.sync_copy(zbuf_v, acc.at[pl.ds(sid * rps, rps)])
        plsc.subcore_barrier()

        pltpu.sync_copy(edge_hbm.at[1, wid], dst_v)

        @pl.loop(0, k)
        def _(j):
            pltpu.sync_copy(ones_v, acc.at[dst_v.at[j]], add=True)

        plsc.subcore_barrier()
        pltpu.sync_copy(
            acc.at[pl.ds(sid * rps, rps)],
            out_hbm.at[cid, pl.ds(sid * rps, rps)],
        )

    return deg_kernel(edge_r)


@functools.partial(jax.jit, static_argnames=("n", "k", "chunk"))
def _gather_scatter_pass(y, edge_r, *, n, k, chunk):
    np_ = _pad_rows(n)
    rps = np_ // NSUB

    stage = n // NSUB
    tail = n - NSUB * stage

    @functools.partial(
        pl.kernel,
        mesh=_mesh,
        compiler_params=_sc_params,
        out_type=jax.ShapeDtypeStruct((NCORES, np_, D), jnp.float32),
        scratch_types=[
            pltpu.VMEM((k, chunk), jnp.int32),
            pltpu.VMEM((k, chunk), jnp.int32),
            pltpu.VMEM((chunk, D), jnp.float32),
            pltpu.VMEM((chunk, D), jnp.float32),
            pltpu.VMEM((rps, D), jnp.float32),
            pltpu.VMEM_SHARED((n, D), jnp.float32),
            pltpu.VMEM_SHARED((np_, D), jnp.float32),
            pltpu.SemaphoreType.DMA,
            pltpu.SemaphoreType.DMA,
        ],
    )
    def scat_kernel(y_hbm, edge_hbm, out_hbm, src_v, dst_v,
                    rows0_v, rows1_v, zbuf_v, y_spm, acc, sem0, sem1):
        cid = lax.axis_index("c")
        sid = lax.axis_index("s")
        wid = cid * NSUB + sid

        pltpu.sync_copy(y_hbm.at[pl.ds(sid * stage, stage)],
                        y_spm.at[pl.ds(sid * stage, stage)])
        if tail:
            @pl.when(sid == 0)
            def _():
                pltpu.sync_copy(y_hbm.at[pl.ds(NSUB * stage, tail)],
                                y_spm.at[pl.ds(NSUB * stage, tail)])

        @pl.loop(0, rps)
        def _(r):
            zbuf_v[r, :] = jnp.zeros((D,), jnp.float32)

        pltpu.sync_copy(zbuf_v, acc.at[pl.ds(sid * rps, rps)])
        plsc.subcore_barrier()

        pltpu.sync_copy(edge_hbm.at[0, wid], src_v)
        pltpu.sync_copy(edge_hbm.at[1, wid], dst_v)

        pltpu.async_copy(y_spm.at[src_v.at[0]], rows0_v, sem0).wait()

        @pl.loop(0, (k - 1) // 2)
        def _(jj):
            j = jj * 2
            pltpu.async_copy(y_spm.at[src_v.at[j + 1]], rows1_v, sem1)
            pltpu.sync_copy(rows0_v, acc.at[dst_v.at[j]], add=True)
            pltpu.make_async_copy(y_spm.at[src_v.at[j + 1]], rows1_v, sem1).wait()
            pltpu.async_copy(y_spm.at[src_v.at[j + 2]], rows0_v, sem0)
            pltpu.sync_copy(rows1_v, acc.at[dst_v.at[j + 1]], add=True)
            pltpu.make_async_copy(y_spm.at[src_v.at[j + 2]], rows0_v, sem0).wait()

        pltpu.sync_copy(rows0_v, acc.at[dst_v.at[k - 1]], add=True)

        plsc.subcore_barrier()
        pltpu.sync_copy(
            acc.at[pl.ds(sid * rps, rps)],
            out_hbm.at[cid, pl.ds(sid * rps, rps)],
        )

    return scat_kernel(y, edge_r)


@functools.partial(jax.jit, static_argnames=("n", "k", "chunk"))
def _fused_pass1(xw, degp, edge_r, *, n, k, chunk):
    np_ = _pad_rows(n)
    rps = np_ // NSUB
    stage = n // NSUB
    tail = n - NSUB * stage
    degstage = -(-stage // D) * D

    @functools.partial(
        pl.kernel,
        mesh=_mesh,
        compiler_params=_sc_params_nl,
        out_type=(
            jax.ShapeDtypeStruct((NCORES, np_, D), jnp.float32),
            jax.ShapeDtypeStruct((n, D), jnp.float32),
            jax.ShapeDtypeStruct((n, D), jnp.float32),
        ),
        scratch_types=[
            pltpu.VMEM((k, chunk), jnp.int32),
            pltpu.VMEM((k, chunk), jnp.int32),
            pltpu.VMEM((chunk, D), jnp.float32),
            pltpu.VMEM((chunk, D), jnp.float32),
            pltpu.VMEM((rps, D), jnp.float32),
            pltpu.VMEM((degstage, D), jnp.float32),
            pltpu.VMEM((degstage, D), jnp.float32),
            pltpu.VMEM((stage + (NSUB * 8), D), jnp.float32),
            pltpu.VMEM((degstage,), jnp.float32),
            pltpu.VMEM_SHARED((n, D), jnp.float32),
            pltpu.VMEM_SHARED((np_, D), jnp.float32),
            pltpu.SemaphoreType.DMA,
            pltpu.SemaphoreType.DMA,
        ],
    )
    def fused_kernel(xw_hbm, degp_hbm, edge_hbm,
                     out_hbm, dis_hbm, dinv_hbm,
                     src_v, dst_v, rows0_v, rows1_v, zbuf_v,
                     d0buf, d1buf, xwbuf, dis_pack, y_spm, acc, sem0, sem1):
        cid = lax.axis_index("c")
        sid = lax.axis_index("s")
        wid = cid * NSUB + sid

        iotav = lax.iota(jnp.int32, D)
        zerovf = jnp.zeros((D,), jnp.float32)

        @pl.loop(0, rps)
        def _(r):
            plsc.store_scatter(zbuf_v, [jnp.full((D,), r, jnp.int32), iotav],
                               zerovf)

        pltpu.sync_copy(zbuf_v, acc.at[pl.ds(sid * rps, rps)])

        base = sid * stage
        pltpu.sync_copy(degp_hbm.at[0, pl.ds(base, degstage)],
                        d0buf.at[pl.ds(0, degstage)])
        pltpu.sync_copy(degp_hbm.at[1, pl.ds(base, degstage)],
                        d1buf.at[pl.ds(0, degstage)])
        pltpu.sync_copy(xw_hbm.at[pl.ds(base, stage)],
                        xwbuf.at[pl.ds(0, stage)])

        onev = jnp.full((D,), 1.0, jnp.float32)
        halfv = jnp.full((D,), 0.5, jnp.float32)
        threehalfv = jnp.full((D,), 1.5, jnp.float32)
        magicv = jnp.full((D,), 0x5F3759DF, jnp.int32)
        shiftv = jnp.full((D,), 1, jnp.int32)
        zerov = jnp.zeros((D,), jnp.int32)

        def row_load(buf, r):
            return plsc.load_gather(buf, [jnp.full((D,), r, jnp.int32), iotav])

        def row_store(buf, r, x):
            plsc.store_scatter(buf, [jnp.full((D,), r, jnp.int32), iotav], x)

        def newton(deg):
            i = lax.bitcast_convert_type(deg, jnp.int32)
            i = magicv - lax.shift_right_arithmetic(i, shiftv)
            ys = lax.bitcast_convert_type(i, jnp.float32)
            for _ in range(3):
                ys = ys * (threehalfv - halfv * deg * ys * ys)
            return ys

        @pl.loop(0, degstage // D)
        def _(v):
            rows = iotav + v * D
            g0 = plsc.load_gather(d0buf, [rows, zerov])
            g1 = plsc.load_gather(d1buf, [rows, zerov])
            dis_pack[pl.ds(v * D, D)] = newton(g0 + g1 + onev)

        @pl.loop(0, stage)
        def _(r):
            splat = plsc.load_gather(dis_pack, [jnp.full((D,), r, jnp.int32)])
            row_store(d0buf, r, row_load(xwbuf, r) * splat)
            row_store(d1buf, r, splat)
            row_store(xwbuf, r, splat * splat)

        if tail:
            @pl.when(sid == 0)
            def _():
                pltpu.sync_copy(degp_hbm.at[0, pl.ds(NSUB * stage, tail)],
                                rows0_v.at[pl.ds(0, tail)])
                pltpu.sync_copy(degp_hbm.at[1, pl.ds(NSUB * stage, tail)],
                                rows1_v.at[pl.ds(0, tail)])
                pltpu.sync_copy(xw_hbm.at[pl.ds(NSUB * stage, tail)],
                                xwbuf.at[pl.ds(stage, tail)])

                @pl.loop(0, tail)
                def _(r):
                    ys = newton(row_load(rows0_v, r) + row_load(rows1_v, r)
                                + onev)
                    row_store(rows0_v, r, row_load(xwbuf, stage + r) * ys)
                    row_store(rows1_v, r, ys)
                    row_store(xwbuf, stage + r, ys * ys)

        pltpu.sync_copy(d0buf.at[pl.ds(0, stage)],
                        y_spm.at[pl.ds(base, stage)])

        @pl.when(cid == 0)
        def _():
            pltpu.sync_copy(d1buf.at[pl.ds(0, stage)],
                            dis_hbm.at[pl.ds(base, stage)])
            pltpu.sync_copy(xwbuf.at[pl.ds(0, stage)],
                            dinv_hbm.at[pl.ds(base, stage)])
        if tail:
            @pl.when(sid == 0)
            def _():
                pltpu.sync_copy(rows0_v.at[pl.ds(0, tail)],
                                y_spm.at[pl.ds(NSUB * stage, tail)])

                @pl.when(cid == 0)
                def _():
                    pltpu.sync_copy(rows1_v.at[pl.ds(0, tail)],
                                    dis_hbm.at[pl.ds(NSUB * stage, tail)])
                    pltpu.sync_copy(xwbuf.at[pl.ds(stage, tail)],
                                    dinv_hbm.at[pl.ds(NSUB * stage, tail)])

        plsc.subcore_barrier()

        pltpu.sync_copy(edge_hbm.at[0, wid], src_v)
        pltpu.sync_copy(edge_hbm.at[1, wid], dst_v)

        pltpu.async_copy(y_spm.at[src_v.at[0]], rows0_v, sem0).wait()

        @pl.loop(0, (k - 1) // 2)
        def _(jj):
            j = jj * 2
            pltpu.async_copy(y_spm.at[src_v.at[j + 1]], rows1_v, sem1)
            pltpu.sync_copy(rows0_v, acc.at[dst_v.at[j]], add=True)
            pltpu.make_async_copy(y_spm.at[src_v.at[j + 1]], rows1_v, sem1).wait()
            pltpu.async_copy(y_spm.at[src_v.at[j + 2]], rows0_v, sem0)
            pltpu.sync_copy(rows1_v, acc.at[dst_v.at[j + 1]], add=True)
            pltpu.make_async_copy(y_spm.at[src_v.at[j + 2]], rows0_v, sem0).wait()

        pltpu.sync_copy(rows0_v, acc.at[dst_v.at[k - 1]], add=True)

        plsc.subcore_barrier()
        pltpu.sync_copy(
            acc.at[pl.ds(sid * rps, rps)],
            out_hbm.at[cid, pl.ds(sid * rps, rps)],
        )

    return fused_kernel(xw, degp, edge_r)


def _mm_body(x_ref, w_ref, o_ref):
    o_ref[...] = jnp.dot(x_ref[...], w_ref[...],
                         preferred_element_type=jnp.float32)


def _make_layer1_body(n8):
    def _layer1_body(sp_ref, xw_ref, dis_ref, dinv_ref, b_ref, w2_ref,
                     y2_ref, hw2_ref):
        dis = dis_ref[...]
        s = sp_ref[0, :n8, :] + sp_ref[1, :n8, :]
        pre = dis * s + xw_ref[...] * dinv_ref[...] + b_ref[...]
        h = jnp.maximum(pre, 0.0)
        hw2 = jnp.dot(h, w2_ref[...], preferred_element_type=jnp.float32)
        hw2_ref[...] = hw2
        y2_ref[...] = hw2 * dis
    return _layer1_body


def _make_layer2_body(n8):
    def _layer2_body(sp_ref, hw2_ref, dis_ref, dinv_ref, b_ref, o_ref):
        s = sp_ref[0, :n8, :] + sp_ref[1, :n8, :]
        o_ref[...] = dis_ref[...] * s + hw2_ref[...] * dinv_ref[...] \
            + b_ref[...]
    return _layer2_body


def _pick_chunk(ept):
    for c in range(128, 7, -8):
        if ept % c == 0 and (ept // c) % 2 == 1 and ept // c >= 3:
            return c
    return None


def kernel(x, edge_index, W1, b1, W2, b2):
    n = x.shape[0]
    e = edge_index.shape[1]
    np_ = _pad_rows(n)

    chunk = _pick_chunk(e // NTILES) if e % NTILES == 0 else None
    if chunk is not None:
        k = e // (NTILES * chunk)
        edge_r = edge_index.reshape(2, NTILES, k, chunk)
    else:
        chunk = CHUNK
        k = -(-e // (NTILES * chunk))
        if k % 2 == 0:
            k += 1
        pad = NTILES * k * chunk - e
        srcp = jnp.concatenate([edge_index[0], jnp.zeros((pad,), jnp.int32)])
        dstp = jnp.concatenate([edge_index[1], jnp.full((pad,), n, jnp.int32)])
        edge_r = jnp.stack([srcp, dstp]).reshape(2, NTILES, k, chunk)

    n8 = n // 8
    fshape = jax.ShapeDtypeStruct((n8, 8 * D), jnp.float32)
    eye8 = jnp.eye(8, dtype=jnp.float32)
    w1bd = jnp.kron(eye8, W1)
    w2bd = jnp.kron(eye8, W2)
    b1t = jnp.tile(b1, 8).reshape(1, 8 * D)
    b2t = jnp.tile(b2, 8).reshape(1, 8 * D)
    xr = x.reshape(n8, 8 * x.shape[1])

    np8 = _pad_rows(n) // 8

    degp = _deg_pass(edge_r, n=n, k=k, chunk=chunk)
    xw1 = pl.pallas_call(_mm_body, out_shape=fshape)(xr, w1bd)

    s1p, dis, dinv = _fused_pass1(xw1.reshape(n, D), degp, edge_r,
                                  n=n, k=k, chunk=chunk)

    y2, hw2 = pl.pallas_call(
        _make_layer1_body(n8), out_shape=(fshape, fshape)
    )(s1p.reshape(NCORES, np8, 8 * D), xw1, dis.reshape(n8, 8 * D),
      dinv.reshape(n8, 8 * D), b1t, w2bd)

    s2p = _gather_scatter_pass(y2.reshape(n, D), edge_r,
                               n=n, k=k, chunk=chunk)

    out = pl.pallas_call(
        _make_layer2_body(n8), out_shape=fshape
    )(s2p.reshape(NCORES, np8, 8 * D), hw2,
      dis.reshape(n8, 8 * D), dinv.reshape(n8, 8 * D), b2t)
    return out.reshape(n, D)


# speedup vs baseline: 79.4871x; 1.0330x over previous
import functools

import jax
import jax.numpy as jnp
from jax import lax
from jax.experimental import pallas as pl
from jax.experimental.pallas import tpu as pltpu
from jax.experimental.pallas import tpu_sc as plsc

D = 16
CHUNK = 128
NCORES = 2
NSUB = 16
NTILES = NCORES * NSUB

_mesh = plsc.VectorSubcoreMesh(core_axis_name="c", subcore_axis_name="s")
_sc_params = pltpu.CompilerParams(use_tc_tiling_on_sc=False)
_sc_params_nl = pltpu.CompilerParams(use_tc_tiling_on_sc=False,
                                     needs_layout_passes=False)


def _pad_rows(n):
    return ((n + NSUB + NSUB * 8 - 1) // (NSUB * 8)) * (NSUB * 8)


@functools.partial(jax.jit, static_argnames=("n", "k", "chunk"))
def _deg_pass(edge_r, *, n, k, chunk):
    np_ = _pad_rows(n)
    rps = np_ // NSUB

    @functools.partial(
        pl.kernel,
        mesh=_mesh,
        compiler_params=_sc_params,
        out_type=jax.ShapeDtypeStruct((NCORES, np_, D), jnp.float32),
        scratch_types=[
            pltpu.VMEM((k * chunk,), jnp.int32),
            pltpu.VMEM((chunk, D), jnp.float32),
            pltpu.VMEM((rps, D), jnp.float32),
            pltpu.VMEM_SHARED((np_, D), jnp.float32),
            pltpu.SemaphoreType.DMA,
        ],
    )
    def deg_kernel(edge_hbm, out_hbm, dst_v, ones_v, zbuf_v, acc, semi):
        cid = lax.axis_index("c")
        sid = lax.axis_index("s")
        wid = cid * NSUB + sid
        ept = k * chunk

        pltpu.async_copy(edge_hbm.at[1, pl.ds(wid * ept, ept)], dst_v, semi)

        @pl.loop(0, rps)
        def _(r):
            zbuf_v[r, :] = jnp.zeros((D,), jnp.float32)

        @pl.loop(0, chunk)
        def _(r):
            ones_v[r, :] = jnp.ones((D,), jnp.float32)

        pltpu.sync_copy(zbuf_v, acc.at[pl.ds(sid * rps, rps)])
        plsc.subcore_barrier()
        pltpu.make_async_copy(edge_hbm.at[1, pl.ds(wid * ept, ept)],
                              dst_v, semi).wait()

        @pl.loop(0, k)
        def _(j):
            idx = dst_v.at[pl.ds(pl.multiple_of(j * chunk, 8), chunk)]
            pltpu.sync_copy(ones_v, acc.at[idx], add=True)

        plsc.subcore_barrier()
        pltpu.sync_copy(
            acc.at[pl.ds(sid * rps, rps)],
            out_hbm.at[cid, pl.ds(sid * rps, rps)],
        )

    return deg_kernel(edge_r)


@functools.partial(jax.jit, static_argnames=("n", "k", "chunk"))
def _gather_scatter_pass(y, edge_r, *, n, k, chunk):
    np_ = _pad_rows(n)
    rps = np_ // NSUB

    stage = n // NSUB
    tail = n - NSUB * stage

    @functools.partial(
        pl.kernel,
        mesh=_mesh,
        compiler_params=_sc_params,
        out_type=jax.ShapeDtypeStruct((NCORES, np_, D), jnp.float32),
        scratch_types=[
            pltpu.VMEM((k * chunk,), jnp.int32),
            pltpu.VMEM((k * chunk,), jnp.int32),
            pltpu.VMEM((chunk, D), jnp.float32),
            pltpu.VMEM((chunk, D), jnp.float32),
            pltpu.VMEM((rps, D), jnp.float32),
            pltpu.VMEM_SHARED((n, D), jnp.float32),
            pltpu.VMEM_SHARED((np_, D), jnp.float32),
            pltpu.SemaphoreType.DMA,
            pltpu.SemaphoreType.DMA,
            pltpu.SemaphoreType.DMA,
        ],
    )
    def scat_kernel(y_hbm, edge_hbm, out_hbm, src_v, dst_v,
                    rows0_v, rows1_v, zbuf_v, y_spm, acc, sem0, sem1, semi):
        cid = lax.axis_index("c")
        sid = lax.axis_index("s")
        wid = cid * NSUB + sid
        ept = k * chunk

        def cs(j, c=chunk):
            return pl.ds(pl.multiple_of(j * c, 8), c)

        pltpu.async_copy(edge_hbm.at[0, pl.ds(wid * ept, ept)], src_v, semi)
        pltpu.async_copy(edge_hbm.at[1, pl.ds(wid * ept, ept)], dst_v, semi)

        pltpu.sync_copy(y_hbm.at[pl.ds(sid * stage, stage)],
                        y_spm.at[pl.ds(sid * stage, stage)])
        if tail:
            @pl.when(sid == 0)
            def _():
                pltpu.sync_copy(y_hbm.at[pl.ds(NSUB * stage, tail)],
                                y_spm.at[pl.ds(NSUB * stage, tail)])

        @pl.loop(0, rps)
        def _(r):
            zbuf_v[r, :] = jnp.zeros((D,), jnp.float32)

        pltpu.sync_copy(zbuf_v, acc.at[pl.ds(sid * rps, rps)])
        plsc.subcore_barrier()

        pltpu.make_async_copy(edge_hbm.at[0, pl.ds(wid * ept, ept)],
                              src_v, semi).wait()
        pltpu.make_async_copy(edge_hbm.at[1, pl.ds(wid * ept, ept)],
                              dst_v, semi).wait()

        pltpu.async_copy(y_spm.at[src_v.at[cs(0)]], rows0_v, sem0).wait()

        @pl.loop(0, (k - 1) // 2)
        def _(jj):
            j = jj * 2
            pltpu.async_copy(y_spm.at[src_v.at[cs(j + 1)]], rows1_v, sem1)
            pltpu.sync_copy(rows0_v, acc.at[dst_v.at[cs(j)]], add=True)
            pltpu.make_async_copy(y_spm.at[src_v.at[cs(j + 1)]],
                                  rows1_v, sem1).wait()
            pltpu.async_copy(y_spm.at[src_v.at[cs(j + 2)]], rows0_v, sem0)
            pltpu.sync_copy(rows1_v, acc.at[dst_v.at[cs(j + 1)]], add=True)
            pltpu.make_async_copy(y_spm.at[src_v.at[cs(j + 2)]],
                                  rows0_v, sem0).wait()

        pltpu.sync_copy(rows0_v, acc.at[dst_v.at[cs(k - 1)]], add=True)

        plsc.subcore_barrier()
        pltpu.sync_copy(
            acc.at[pl.ds(sid * rps, rps)],
            out_hbm.at[cid, pl.ds(sid * rps, rps)],
        )

    return scat_kernel(y, edge_r)


@functools.partial(jax.jit, static_argnames=("n", "k", "chunk"))
def _fused_pass1(xw, degp, edge_r, *, n, k, chunk):
    np_ = _pad_rows(n)
    rps = np_ // NSUB
    stage = n // NSUB
    tail = n - NSUB * stage
    degstage = -(-stage // D) * D

    @functools.partial(
        pl.kernel,
        mesh=_mesh,
        compiler_params=_sc_params_nl,
        out_type=(
            jax.ShapeDtypeStruct((NCORES, np_, D), jnp.float32),
            jax.ShapeDtypeStruct((n, D), jnp.float32),
            jax.ShapeDtypeStruct((n, D), jnp.float32),
        ),
        scratch_types=[
            pltpu.VMEM((k * chunk,), jnp.int32),
            pltpu.VMEM((k * chunk,), jnp.int32),
            pltpu.VMEM((chunk, D), jnp.float32),
            pltpu.VMEM((chunk, D), jnp.float32),
            pltpu.VMEM((rps, D), jnp.float32),
            pltpu.VMEM((degstage, D), jnp.float32),
            pltpu.VMEM((degstage, D), jnp.float32),
            pltpu.VMEM((stage + (NSUB * 8), D), jnp.float32),
            pltpu.VMEM((degstage,), jnp.float32),
            pltpu.VMEM_SHARED((n, D), jnp.float32),
            pltpu.VMEM_SHARED((np_, D), jnp.float32),
            pltpu.SemaphoreType.DMA,
            pltpu.SemaphoreType.DMA,
        ],
    )
    def fused_kernel(xw_hbm, degp_hbm, edge_hbm,
                     out_hbm, dis_hbm, dinv_hbm,
                     src_v, dst_v, rows0_v, rows1_v, zbuf_v,
                     d0buf, d1buf, xwbuf, dis_pack, y_spm, acc, sem0, sem1):
        cid = lax.axis_index("c")
        sid = lax.axis_index("s")
        wid = cid * NSUB + sid
        ept = k * chunk

        def cs(j, c=chunk):
            return pl.ds(pl.multiple_of(j * c, 8), c)

        iotav = lax.iota(jnp.int32, D)
        zerovf = jnp.zeros((D,), jnp.float32)

        @pl.loop(0, rps)
        def _(r):
            plsc.store_scatter(zbuf_v, [jnp.full((D,), r, jnp.int32), iotav],
                               zerovf)

        pltpu.sync_copy(zbuf_v, acc.at[pl.ds(sid * rps, rps)])

        base = sid * stage
        pltpu.sync_copy(degp_hbm.at[0, pl.ds(base, degstage)],
                        d0buf.at[pl.ds(0, degstage)])
        pltpu.sync_copy(degp_hbm.at[1, pl.ds(base, degstage)],
                        d1buf.at[pl.ds(0, degstage)])
        pltpu.sync_copy(xw_hbm.at[pl.ds(base, stage)],
                        xwbuf.at[pl.ds(0, stage)])

        onev = jnp.full((D,), 1.0, jnp.float32)
        halfv = jnp.full((D,), 0.5, jnp.float32)
        threehalfv = jnp.full((D,), 1.5, jnp.float32)
        magicv = jnp.full((D,), 0x5F3759DF, jnp.int32)
        shiftv = jnp.full((D,), 1, jnp.int32)
        zerov = jnp.zeros((D,), jnp.int32)

        def row_load(buf, r):
            return plsc.load_gather(buf, [jnp.full((D,), r, jnp.int32), iotav])

        def row_store(buf, r, x):
            plsc.store_scatter(buf, [jnp.full((D,), r, jnp.int32), iotav], x)

        def newton(deg):
            i = lax.bitcast_convert_type(deg, jnp.int32)
            i = magicv - lax.shift_right_arithmetic(i, shiftv)
            ys = lax.bitcast_convert_type(i, jnp.float32)
            for _ in range(3):
                ys = ys * (threehalfv - halfv * deg * ys * ys)
            return ys

        @pl.loop(0, degstage // D)
        def _(v):
            rows = iotav + v * D
            g0 = plsc.load_gather(d0buf, [rows, zerov])
            g1 = plsc.load_gather(d1buf, [rows, zerov])
            dis_pack[pl.ds(v * D, D)] = newton(g0 + g1 + onev)

        @pl.loop(0, stage)
        def _(r):
            splat = plsc.load_gather(dis_pack, [jnp.full((D,), r, jnp.int32)])
            row_store(d0buf, r, row_load(xwbuf, r) * splat)
            row_store(d1buf, r, splat)
            row_store(xwbuf, r, splat * splat)

        if tail:
            @pl.when(sid == 0)
            def _():
                pltpu.sync_copy(degp_hbm.at[0, pl.ds(NSUB * stage, tail)],
                                rows0_v.at[pl.ds(0, tail)])
                pltpu.sync_copy(degp_hbm.at[1, pl.ds(NSUB * stage, tail)],
                                rows1_v.at[pl.ds(0, tail)])
                pltpu.sync_copy(xw_hbm.at[pl.ds(NSUB * stage, tail)],
                                xwbuf.at[pl.ds(stage, tail)])

                @pl.loop(0, tail)
                def _(r):
                    ys = newton(row_load(rows0_v, r) + row_load(rows1_v, r)
                                + onev)
                    row_store(rows0_v, r, row_load(xwbuf, stage + r) * ys)
                    row_store(rows1_v, r, ys)
                    row_store(xwbuf, stage + r, ys * ys)

        pltpu.sync_copy(d0buf.at[pl.ds(0, stage)],
                        y_spm.at[pl.ds(base, stage)])

        @pl.when(cid == 0)
        def _():
            pltpu.sync_copy(d1buf.at[pl.ds(0, stage)],
                            dis_hbm.at[pl.ds(base, stage)])
            pltpu.sync_copy(xwbuf.at[pl.ds(0, stage)],
                            dinv_hbm.at[pl.ds(base, stage)])
        if tail:
            @pl.when(sid == 0)
            def _():
                pltpu.sync_copy(rows0_v.at[pl.ds(0, tail)],
                                y_spm.at[pl.ds(NSUB * stage, tail)])

                @pl.when(cid == 0)
                def _():
                    pltpu.sync_copy(rows1_v.at[pl.ds(0, tail)],
                                    dis_hbm.at[pl.ds(NSUB * stage, tail)])
                    pltpu.sync_copy(xwbuf.at[pl.ds(stage, tail)],
                                    dinv_hbm.at[pl.ds(NSUB * stage, tail)])

        plsc.subcore_barrier()

        pltpu.sync_copy(edge_hbm.at[0, pl.ds(wid * ept, ept)], src_v)
        pltpu.sync_copy(edge_hbm.at[1, pl.ds(wid * ept, ept)], dst_v)

        pltpu.async_copy(y_spm.at[src_v.at[cs(0)]], rows0_v, sem0).wait()

        @pl.loop(0, (k - 1) // 2)
        def _(jj):
            j = jj * 2
            pltpu.async_copy(y_spm.at[src_v.at[cs(j + 1)]], rows1_v, sem1)
            pltpu.sync_copy(rows0_v, acc.at[dst_v.at[cs(j)]], add=True)
            pltpu.make_async_copy(y_spm.at[src_v.at[cs(j + 1)]],
                                  rows1_v, sem1).wait()
            pltpu.async_copy(y_spm.at[src_v.at[cs(j + 2)]], rows0_v, sem0)
            pltpu.sync_copy(rows1_v, acc.at[dst_v.at[cs(j + 1)]], add=True)
            pltpu.make_async_copy(y_spm.at[src_v.at[cs(j + 2)]],
                                  rows0_v, sem0).wait()

        pltpu.sync_copy(rows0_v, acc.at[dst_v.at[cs(k - 1)]], add=True)

        plsc.subcore_barrier()
        pltpu.sync_copy(
            acc.at[pl.ds(sid * rps, rps)],
            out_hbm.at[cid, pl.ds(sid * rps, rps)],
        )

    return fused_kernel(xw, degp, edge_r)


def _mm_body(x_ref, w_ref, o_ref):
    o_ref[...] = jnp.dot(x_ref[...], w_ref[...],
                         preferred_element_type=jnp.float32)


def _make_layer1_body(n8):
    def _layer1_body(sp_ref, xw_ref, dis_ref, dinv_ref, b_ref, w2_ref,
                     y2_ref, hw2_ref):
        dis = dis_ref[...]
        s = sp_ref[0, :n8, :] + sp_ref[1, :n8, :]
        pre = dis * s + xw_ref[...] * dinv_ref[...] + b_ref[...]
        h = jnp.maximum(pre, 0.0)
        hw2 = jnp.dot(h, w2_ref[...], preferred_element_type=jnp.float32)
        hw2_ref[...] = hw2
        y2_ref[...] = hw2 * dis
    return _layer1_body


def _make_layer2_body(n8):
    def _layer2_body(sp_ref, hw2_ref, dis_ref, dinv_ref, b_ref, o_ref):
        s = sp_ref[0, :n8, :] + sp_ref[1, :n8, :]
        o_ref[...] = dis_ref[...] * s + hw2_ref[...] * dinv_ref[...] \
            + b_ref[...]
    return _layer2_body


def _pick_chunk(ept):
    for c in range(128, 7, -8):
        if ept % c == 0 and (ept // c) % 2 == 1 and ept // c >= 3:
            return c
    return None


def kernel(x, edge_index, W1, b1, W2, b2):
    n = x.shape[0]
    e = edge_index.shape[1]
    np_ = _pad_rows(n)

    chunk = _pick_chunk(e // NTILES) if e % NTILES == 0 else None
    if chunk is not None:
        k = e // (NTILES * chunk)
        edge_r = edge_index
    else:
        chunk = CHUNK
        k = -(-e // (NTILES * chunk))
        if k % 2 == 0:
            k += 1
        pad = NTILES * k * chunk - e
        srcp = jnp.concatenate([edge_index[0], jnp.zeros((pad,), jnp.int32)])
        dstp = jnp.concatenate([edge_index[1], jnp.full((pad,), n, jnp.int32)])
        edge_r = jnp.stack([srcp, dstp])

    n8 = n // 8
    fshape = jax.ShapeDtypeStruct((n8, 8 * D), jnp.float32)
    eye8 = jnp.eye(8, dtype=jnp.float32)
    w1bd = jnp.kron(eye8, W1)
    w2bd = jnp.kron(eye8, W2)
    b1t = jnp.tile(b1, 8).reshape(1, 8 * D)
    b2t = jnp.tile(b2, 8).reshape(1, 8 * D)
    xr = x.reshape(n8, 8 * x.shape[1])

    np8 = _pad_rows(n) // 8

    degp = _deg_pass(edge_r, n=n, k=k, chunk=chunk)
    xw1 = pl.pallas_call(_mm_body, out_shape=fshape)(xr, w1bd)

    s1p, dis, dinv = _fused_pass1(xw1.reshape(n, D), degp, edge_r,
                                  n=n, k=k, chunk=chunk)

    y2, hw2 = pl.pallas_call(
        _make_layer1_body(n8), out_shape=(fshape, fshape)
    )(s1p.reshape(NCORES, np8, 8 * D), xw1, dis.reshape(n8, 8 * D),
      dinv.reshape(n8, 8 * D), b1t, w2bd)

    s2p = _gather_scatter_pass(y2.reshape(n, D), edge_r,
                               n=n, k=k, chunk=chunk)

    out = pl.pallas_call(
        _make_layer2_body(n8), out_shape=fshape
    )(s2p.reshape(NCORES, np8, 8 * D), hw2,
      dis.reshape(n8, 8 * D), dinv.reshape(n8, 8 * D), b2t)
    return out.reshape(n, D)


# speedup vs baseline: 84.7100x; 1.0657x over previous
import functools

import jax
import jax.numpy as jnp
from jax import lax
from jax.experimental import pallas as pl
from jax.experimental.pallas import tpu as pltpu
from jax.experimental.pallas import tpu_sc as plsc

D = 16
CHUNK = 128
NCORES = 2
NSUB = 16
NTILES = NCORES * NSUB

_mesh = plsc.VectorSubcoreMesh(core_axis_name="c", subcore_axis_name="s")
_sc_params = pltpu.CompilerParams(use_tc_tiling_on_sc=False)
_sc_params_nl = pltpu.CompilerParams(use_tc_tiling_on_sc=False,
                                     needs_layout_passes=False)


def _pad_deg(n):
    return ((n + NSUB + 64 + 255) // 256) * 256


def _pad_rows(n):
    return ((n + NSUB + NSUB * 8 - 1) // (NSUB * 8)) * (NSUB * 8)


@functools.partial(jax.jit, static_argnames=("n", "k", "chunk"))
def _deg_pass(edge_r, *, n, k, chunk):
    np1 = _pad_deg(n)
    rps = np1 // NSUB

    @functools.partial(
        pl.kernel,
        mesh=_mesh,
        compiler_params=_sc_params,
        out_type=jax.ShapeDtypeStruct((NCORES, np1), jnp.float32),
        scratch_types=[
            pltpu.VMEM((k * chunk,), jnp.int32),
            pltpu.VMEM((chunk,), jnp.float32),
            pltpu.VMEM((rps,), jnp.float32),
            pltpu.VMEM_SHARED((np1,), jnp.float32),
            pltpu.SemaphoreType.DMA,
        ],
    )
    def deg_kernel(edge_hbm, out_hbm, dst_v, ones_v, zbuf_v, acc, semi):
        cid = lax.axis_index("c")
        sid = lax.axis_index("s")
        wid = cid * NSUB + sid
        ept = k * chunk

        pltpu.async_copy(edge_hbm.at[1, pl.ds(wid * ept, ept)], dst_v, semi)

        zv = jnp.zeros((D,), jnp.float32)

        @pl.loop(0, rps // D)
        def _(r):
            zbuf_v[pl.ds(r * D, D)] = zv

        ov = jnp.ones((D,), jnp.float32)

        @pl.loop(0, chunk // D)
        def _(r):
            ones_v[pl.ds(r * D, D)] = ov

        pltpu.sync_copy(zbuf_v, acc.at[pl.ds(sid * rps, rps)])
        plsc.subcore_barrier()
        pltpu.make_async_copy(edge_hbm.at[1, pl.ds(wid * ept, ept)],
                              dst_v, semi).wait()

        @pl.loop(0, k)
        def _(j):
            idx = dst_v.at[pl.ds(pl.multiple_of(j * chunk, 8), chunk)]
            pltpu.sync_copy(ones_v, acc.at[idx], add=True)

        plsc.subcore_barrier()
        pltpu.sync_copy(
            acc.at[pl.ds(sid * rps, rps)],
            out_hbm.at[cid, pl.ds(sid * rps, rps)],
        )

    return deg_kernel(edge_r)


@functools.partial(jax.jit, static_argnames=("n", "k", "chunk"))
def _gather_scatter_pass(y, edge_r, *, n, k, chunk):
    np_ = _pad_rows(n)
    rps = np_ // NSUB

    stage = n // NSUB
    tail = n - NSUB * stage

    @functools.partial(
        pl.kernel,
        mesh=_mesh,
        compiler_params=_sc_params,
        out_type=jax.ShapeDtypeStruct((NCORES, np_, D), jnp.float32),
        scratch_types=[
            pltpu.VMEM((k * chunk,), jnp.int32),
            pltpu.VMEM((k * chunk,), jnp.int32),
            pltpu.VMEM((chunk, D), jnp.float32),
            pltpu.VMEM((chunk, D), jnp.float32),
            pltpu.VMEM((rps, D), jnp.float32),
            pltpu.VMEM_SHARED((n, D), jnp.float32),
            pltpu.VMEM_SHARED((np_, D), jnp.float32),
            pltpu.SemaphoreType.DMA,
            pltpu.SemaphoreType.DMA,
            pltpu.SemaphoreType.DMA,
        ],
    )
    def scat_kernel(y_hbm, edge_hbm, out_hbm, src_v, dst_v,
                    rows0_v, rows1_v, zbuf_v, y_spm, acc, sem0, sem1, semi):
        cid = lax.axis_index("c")
        sid = lax.axis_index("s")
        wid = cid * NSUB + sid
        ept = k * chunk

        def cs(j, c=chunk):
            return pl.ds(pl.multiple_of(j * c, 8), c)

        pltpu.async_copy(edge_hbm.at[0, pl.ds(wid * ept, ept)], src_v, semi)
        pltpu.async_copy(edge_hbm.at[1, pl.ds(wid * ept, ept)], dst_v, semi)

        pltpu.sync_copy(y_hbm.at[pl.ds(sid * stage, stage)],
                        y_spm.at[pl.ds(sid * stage, stage)])
        if tail:
            @pl.when(sid == 0)
            def _():
                pltpu.sync_copy(y_hbm.at[pl.ds(NSUB * stage, tail)],
                                y_spm.at[pl.ds(NSUB * stage, tail)])

        @pl.loop(0, rps)
        def _(r):
            zbuf_v[r, :] = jnp.zeros((D,), jnp.float32)

        pltpu.sync_copy(zbuf_v, acc.at[pl.ds(sid * rps, rps)])
        plsc.subcore_barrier()

        pltpu.make_async_copy(edge_hbm.at[0, pl.ds(wid * ept, ept)],
                              src_v, semi).wait()
        pltpu.make_async_copy(edge_hbm.at[1, pl.ds(wid * ept, ept)],
                              dst_v, semi).wait()

        pltpu.async_copy(y_spm.at[src_v.at[cs(0)]], rows0_v, sem0).wait()

        @pl.loop(0, (k - 1) // 2)
        def _(jj):
            j = jj * 2
            pltpu.async_copy(y_spm.at[src_v.at[cs(j + 1)]], rows1_v, sem1)
            pltpu.sync_copy(rows0_v, acc.at[dst_v.at[cs(j)]], add=True)
            pltpu.make_async_copy(y_spm.at[src_v.at[cs(j + 1)]],
                                  rows1_v, sem1).wait()
            pltpu.async_copy(y_spm.at[src_v.at[cs(j + 2)]], rows0_v, sem0)
            pltpu.sync_copy(rows1_v, acc.at[dst_v.at[cs(j + 1)]], add=True)
            pltpu.make_async_copy(y_spm.at[src_v.at[cs(j + 2)]],
                                  rows0_v, sem0).wait()

        pltpu.sync_copy(rows0_v, acc.at[dst_v.at[cs(k - 1)]], add=True)

        plsc.subcore_barrier()
        pltpu.sync_copy(
            acc.at[pl.ds(sid * rps, rps)],
            out_hbm.at[cid, pl.ds(sid * rps, rps)],
        )

    return scat_kernel(y, edge_r)


@functools.partial(jax.jit, static_argnames=("n", "k", "chunk"))
def _fused_pass1(xw, degp, edge_r, *, n, k, chunk):
    np_ = _pad_rows(n)
    rps = np_ // NSUB
    stage = n // NSUB
    tail = n - NSUB * stage
    degstage = -(-stage // D) * D
    np1 = _pad_deg(n)
    pk = degstage + 2 * D

    @functools.partial(
        pl.kernel,
        mesh=_mesh,
        compiler_params=_sc_params_nl,
        out_type=(
            jax.ShapeDtypeStruct((NCORES, np_, D), jnp.float32),
            jax.ShapeDtypeStruct((n, D), jnp.float32),
            jax.ShapeDtypeStruct((n, D), jnp.float32),
        ),
        scratch_types=[
            pltpu.VMEM((k * chunk,), jnp.int32),
            pltpu.VMEM((k * chunk,), jnp.int32),
            pltpu.VMEM((chunk, D), jnp.float32),
            pltpu.VMEM((chunk, D), jnp.float32),
            pltpu.VMEM((rps, D), jnp.float32),
            pltpu.VMEM((degstage, D), jnp.float32),
            pltpu.VMEM((degstage, D), jnp.float32),
            pltpu.VMEM((stage + (NSUB * 8), D), jnp.float32),
            pltpu.VMEM((pk,), jnp.float32),
            pltpu.VMEM((pk,), jnp.float32),
            pltpu.VMEM((pk,), jnp.float32),
            pltpu.VMEM_SHARED((n, D), jnp.float32),
            pltpu.VMEM_SHARED((np_, D), jnp.float32),
            pltpu.SemaphoreType.DMA,
            pltpu.SemaphoreType.DMA,
        ],
    )
    def fused_kernel(xw_hbm, degp_hbm, edge_hbm,
                     out_hbm, dis_hbm, dinv_hbm,
                     src_v, dst_v, rows0_v, rows1_v, zbuf_v,
                     d0buf, d1buf, xwbuf, d0p, d1p, dis_pack,
                     y_spm, acc, sem0, sem1):
        cid = lax.axis_index("c")
        sid = lax.axis_index("s")
        wid = cid * NSUB + sid
        ept = k * chunk

        def cs(j, c=chunk):
            return pl.ds(pl.multiple_of(j * c, 8), c)

        iotav = lax.iota(jnp.int32, D)
        zerovf = jnp.zeros((D,), jnp.float32)

        @pl.loop(0, rps)
        def _(r):
            plsc.store_scatter(zbuf_v, [jnp.full((D,), r, jnp.int32), iotav],
                               zerovf)

        pltpu.sync_copy(zbuf_v, acc.at[pl.ds(sid * rps, rps)])

        base = sid * stage
        lo = pl.multiple_of((base // 8) * 8, 8)
        off = base - lo
        pltpu.sync_copy(degp_hbm.at[0, pl.ds(lo, degstage)],
                        d0p.at[pl.ds(0, degstage)])
        pltpu.sync_copy(degp_hbm.at[1, pl.ds(lo, degstage)],
                        d1p.at[pl.ds(0, degstage)])
        pltpu.sync_copy(xw_hbm.at[pl.ds(base, stage)],
                        xwbuf.at[pl.ds(0, stage)])

        onev = jnp.full((D,), 1.0, jnp.float32)
        halfv = jnp.full((D,), 0.5, jnp.float32)
        threehalfv = jnp.full((D,), 1.5, jnp.float32)
        magicv = jnp.full((D,), 0x5F3759DF, jnp.int32)
        shiftv = jnp.full((D,), 1, jnp.int32)
        zerov = jnp.zeros((D,), jnp.int32)

        def row_load(buf, r):
            return plsc.load_gather(buf, [jnp.full((D,), r, jnp.int32), iotav])

        def row_store(buf, r, x):
            plsc.store_scatter(buf, [jnp.full((D,), r, jnp.int32), iotav], x)

        def newton(deg):
            i = lax.bitcast_convert_type(deg, jnp.int32)
            i = magicv - lax.shift_right_arithmetic(i, shiftv)
            ys = lax.bitcast_convert_type(i, jnp.float32)
            for _ in range(3):
                ys = ys * (threehalfv - halfv * deg * ys * ys)
            return ys

        @pl.loop(0, degstage // D)
        def _(v):
            g0 = d0p[pl.ds(v * D, D)]
            g1 = d1p[pl.ds(v * D, D)]
            dis_pack[pl.ds(v * D, D)] = newton(g0 + g1 + onev)

        @pl.loop(0, stage)
        def _(r):
            splat = plsc.load_gather(dis_pack,
                                     [jnp.full((D,), off + r, jnp.int32)])
            row_store(d0buf, r, row_load(xwbuf, r) * splat)
            row_store(d1buf, r, splat)
            row_store(xwbuf, r, splat * splat)

        if tail:
            @pl.when(sid == 0)
            def _():
                lo2 = pl.multiple_of(((NSUB * stage) // 8) * 8, 8)
                off2 = NSUB * stage - lo2
                pltpu.sync_copy(degp_hbm.at[0, pl.ds(lo2, 2 * D)],
                                d0p.at[pl.ds(degstage, 2 * D)])
                pltpu.sync_copy(degp_hbm.at[1, pl.ds(lo2, 2 * D)],
                                d1p.at[pl.ds(degstage, 2 * D)])
                pltpu.sync_copy(xw_hbm.at[pl.ds(NSUB * stage, tail)],
                                xwbuf.at[pl.ds(stage, tail)])

                @pl.loop(0, 2)
                def _(w):
                    g0 = d0p[pl.ds(degstage + w * D, D)]
                    g1 = d1p[pl.ds(degstage + w * D, D)]
                    dis_pack[pl.ds(degstage + w * D, D)] = \
                        newton(g0 + g1 + onev)

                @pl.loop(0, tail)
                def _(r):
                    splat = plsc.load_gather(
                        dis_pack,
                        [jnp.full((D,), degstage + off2 + r, jnp.int32)])
                    row_store(rows0_v, r, row_load(xwbuf, stage + r) * splat)
                    row_store(rows1_v, r, splat)
                    row_store(xwbuf, stage + r, splat * splat)

        pltpu.sync_copy(d0buf.at[pl.ds(0, stage)],
                        y_spm.at[pl.ds(base, stage)])

        @pl.when(cid == 0)
        def _():
            pltpu.sync_copy(d1buf.at[pl.ds(0, stage)],
                            dis_hbm.at[pl.ds(base, stage)])
            pltpu.sync_copy(xwbuf.at[pl.ds(0, stage)],
                            dinv_hbm.at[pl.ds(base, stage)])
        if tail:
            @pl.when(sid == 0)
            def _():
                pltpu.sync_copy(rows0_v.at[pl.ds(0, tail)],
                                y_spm.at[pl.ds(NSUB * stage, tail)])

                @pl.when(cid == 0)
                def _():
                    pltpu.sync_copy(rows1_v.at[pl.ds(0, tail)],
                                    dis_hbm.at[pl.ds(NSUB * stage, tail)])
                    pltpu.sync_copy(xwbuf.at[pl.ds(stage, tail)],
                                    dinv_hbm.at[pl.ds(NSUB * stage, tail)])

        plsc.subcore_barrier()

        pltpu.sync_copy(edge_hbm.at[0, pl.ds(wid * ept, ept)], src_v)
        pltpu.sync_copy(edge_hbm.at[1, pl.ds(wid * ept, ept)], dst_v)

        pltpu.async_copy(y_spm.at[src_v.at[cs(0)]], rows0_v, sem0).wait()

        @pl.loop(0, (k - 1) // 2)
        def _(jj):
            j = jj * 2
            pltpu.async_copy(y_spm.at[src_v.at[cs(j + 1)]], rows1_v, sem1)
            pltpu.sync_copy(rows0_v, acc.at[dst_v.at[cs(j)]], add=True)
            pltpu.make_async_copy(y_spm.at[src_v.at[cs(j + 1)]],
                                  rows1_v, sem1).wait()
            pltpu.async_copy(y_spm.at[src_v.at[cs(j + 2)]], rows0_v, sem0)
            pltpu.sync_copy(rows1_v, acc.at[dst_v.at[cs(j + 1)]], add=True)
            pltpu.make_async_copy(y_spm.at[src_v.at[cs(j + 2)]],
                                  rows0_v, sem0).wait()

        pltpu.sync_copy(rows0_v, acc.at[dst_v.at[cs(k - 1)]], add=True)

        plsc.subcore_barrier()
        pltpu.sync_copy(
            acc.at[pl.ds(sid * rps, rps)],
            out_hbm.at[cid, pl.ds(sid * rps, rps)],
        )

    return fused_kernel(xw, degp, edge_r)


def _mm_body(x_ref, w_ref, o_ref):
    o_ref[...] = jnp.dot(x_ref[...], w_ref[...],
                         preferred_element_type=jnp.float32)


def _make_layer1_body(n8):
    def _layer1_body(sp_ref, xw_ref, dis_ref, dinv_ref, b_ref, w2_ref,
                     y2_ref, hw2_ref):
        dis = dis_ref[...]
        s = sp_ref[0, :n8, :] + sp_ref[1, :n8, :]
        pre = dis * s + xw_ref[...] * dinv_ref[...] + b_ref[...]
        h = jnp.maximum(pre, 0.0)
        hw2 = jnp.dot(h, w2_ref[...], preferred_element_type=jnp.float32)
        hw2_ref[...] = hw2
        y2_ref[...] = hw2 * dis
    return _layer1_body


def _make_layer2_body(n8):
    def _layer2_body(sp_ref, hw2_ref, dis_ref, dinv_ref, b_ref, o_ref):
        s = sp_ref[0, :n8, :] + sp_ref[1, :n8, :]
        o_ref[...] = dis_ref[...] * s + hw2_ref[...] * dinv_ref[...] \
            + b_ref[...]
    return _layer2_body


def _pick_chunk(ept):
    for c in range(128, 7, -8):
        if ept % c == 0 and (ept // c) % 2 == 1 and ept // c >= 3:
            return c
    return None


def kernel(x, edge_index, W1, b1, W2, b2):
    n = x.shape[0]
    e = edge_index.shape[1]
    np_ = _pad_rows(n)

    chunk = _pick_chunk(e // NTILES) if e % NTILES == 0 else None
    if chunk is not None:
        k = e // (NTILES * chunk)
        edge_r = edge_index
    else:
        chunk = CHUNK
        k = -(-e // (NTILES * chunk))
        if k % 2 == 0:
            k += 1
        pad = NTILES * k * chunk - e
        srcp = jnp.concatenate([edge_index[0], jnp.zeros((pad,), jnp.int32)])
        dstp = jnp.concatenate([edge_index[1], jnp.full((pad,), n, jnp.int32)])
        edge_r = jnp.stack([srcp, dstp])

    n8 = n // 8
    fshape = jax.ShapeDtypeStruct((n8, 8 * D), jnp.float32)
    eye8 = jnp.eye(8, dtype=jnp.float32)
    w1bd = jnp.kron(eye8, W1)
    w2bd = jnp.kron(eye8, W2)
    b1t = jnp.tile(b1, 8).reshape(1, 8 * D)
    b2t = jnp.tile(b2, 8).reshape(1, 8 * D)
    xr = x.reshape(n8, 8 * x.shape[1])

    np8 = _pad_rows(n) // 8

    degp = _deg_pass(edge_r, n=n, k=k, chunk=chunk)
    xw1 = pl.pallas_call(_mm_body, out_shape=fshape)(xr, w1bd)

    s1p, dis, dinv = _fused_pass1(xw1.reshape(n, D), degp, edge_r,
                                  n=n, k=k, chunk=chunk)

    y2, hw2 = pl.pallas_call(
        _make_layer1_body(n8), out_shape=(fshape, fshape)
    )(s1p.reshape(NCORES, np8, 8 * D), xw1, dis.reshape(n8, 8 * D),
      dinv.reshape(n8, 8 * D), b1t, w2bd)

    s2p = _gather_scatter_pass(y2.reshape(n, D), edge_r,
                               n=n, k=k, chunk=chunk)

    out = pl.pallas_call(
        _make_layer2_body(n8), out_shape=fshape
    )(s2p.reshape(NCORES, np8, 8 * D), hw2,
      dis.reshape(n8, 8 * D), dinv.reshape(n8, 8 * D), b2t)
    return out.reshape(n, D)
